# Initial kernel scaffold; baseline (speedup 1.0000x reference)
import jax
import jax.numpy as jnp
from jax.experimental import pallas as pl


def kernel(x, edge_index, batch, Wpre1, bpre1, Wpost1, bpost1, Wlin1, blin1, Wpre2, bpre2, Wpost2, bpost2, Wlin2, blin2, Wfc, bfc):
    raise NotImplementedError("write your pallas kernel here")


# speedup vs baseline: 5.3784x; 5.3784x over previous
import functools
import numpy as np
import jax
import jax.numpy as jnp
from jax import lax
from jax.experimental import pallas as pl
from jax.experimental.pallas import tpu as pltpu
from jax.experimental.pallas import tpu_sc as plsc

N = 100000
E = 1600000
G = 128
AVG_DEG_LOG = float(np.log(17.0))

NC = 2
NS = 16
NW = NC * NS
BW = 784
NB = 128
NP = BW * NB
PASSES = NB // NW

EW = E // NW
WH = 2000
NWIN_H = EW // WH

EP = 1602048
TRASH = EP - 64

WE = 256
GC = WE // 128

_SC_PARAMS = pltpu.CompilerParams(
    needs_layout_passes=False, use_tc_tiling_on_sc=False)


def _mesh():
  return plsc.VectorSubcoreMesh(core_axis_name="c", subcore_axis_name="s")


def _wid():
  return lax.axis_index("s") * NC + lax.axis_index("c")


def _i16():
  return lax.iota(jnp.int32, 16)


def _al(v, m=8):
  return pl.multiple_of(v, m)


@functools.partial(
    pl.kernel, mesh=_mesh(), compiler_params=_SC_PARAMS,
    out_type=jax.ShapeDtypeStruct((NW * NB,), jnp.int32),
    scratch_types=[pltpu.VMEM((WH,), jnp.int32),
                   pltpu.VMEM((NB,), jnp.int32)],
)
def _sc_hist(dst_hbm, hist_hbm, dstw, histv):
  wid = _wid()
  for j in range(NB // 16):
    histv[pl.ds(j * 16, 16)] = jnp.zeros((16,), jnp.int32)
  ones = jnp.ones((16,), jnp.int32)

  def win(w, carry):
    pltpu.sync_copy(dst_hbm.at[pl.ds(_al(wid * EW + w * WH), WH)], dstw)
    for j in range(WH // 16):
      v = dstw[pl.ds(_al(j * 16, 16), 16)]
      b = lax.div(v, BW)
      plsc.addupdate_scatter(histv, [b], ones)
    return carry

  lax.fori_loop(0, NWIN_H, win, 0)
  pltpu.sync_copy(histv, hist_hbm.at[pl.ds(_al(wid * NB), NB)])


@functools.partial(
    pl.kernel, mesh=_mesh(), compiler_params=_SC_PARAMS,
    out_type=[jax.ShapeDtypeStruct((EP,), jnp.int32),
              jax.ShapeDtypeStruct((NB,), jnp.int32),
              jax.ShapeDtypeStruct((NB,), jnp.int32)],
    scratch_types=[pltpu.VMEM((WH,), jnp.int32),
                   pltpu.VMEM((WH,), jnp.int32),
                   pltpu.VMEM((NW * NB,), jnp.int32),
                   pltpu.VMEM((NB,), jnp.int32),
                   pltpu.VMEM((NB,), jnp.int32),
                   pltpu.VMEM((NB,), jnp.int32),
                   pltpu.VMEM((WH,), jnp.int32),
                   pltpu.VMEM((WH,), jnp.int32),
                   pltpu.VMEM((16,), jnp.int32),
                   pltpu.VMEM((16,), jnp.int32),
                   pltpu.SemaphoreType.DMA],
)
def _sc_permute(src_hbm, dst_hbm, hist_hbm, words_hbm, bbase_hbm, bcnt_hbm,
                srcw, dstw, histall, cntv, basev, cur, wobuf, pobuf,
                tag16, pos16, sem):
  wid = _wid()
  i16 = _i16()
  pltpu.sync_copy(hist_hbm, histall)

  carry = jnp.int32(0)
  for bb in range(NB // 16):
    tot = jnp.zeros((16,), jnp.int32)
    for w in range(NW):
      tot = tot + histall[pl.ds(_al(w * NB + bb * 16, 16), 16)]
    cntv[pl.ds(bb * 16, 16)] = tot
    a8 = (tot + 7) & ~7
    incl = plsc.cumsum(a8)
    basev[pl.ds(bb * 16, 16)] = carry + incl - a8
    carry = carry + jnp.sum(a8)

    def add_prev(w, s16):
      return s16 + histall[pl.ds(_al(w * NB + bb * 16, 16), 16)]

    mine = lax.fori_loop(0, wid, add_prev, basev[pl.ds(bb * 16, 16)])
    cur[pl.ds(bb * 16, 16)] = mine

  @pl.when(wid == 0)
  def _():
    pltpu.sync_copy(basev, bbase_hbm)
    pltpu.sync_copy(cntv, bcnt_hbm)

  def win(w, carry2):
    pltpu.sync_copy(src_hbm.at[pl.ds(_al(wid * EW + w * WH), WH)], srcw)
    pltpu.sync_copy(dst_hbm.at[pl.ds(_al(wid * EW + w * WH), WH)], dstw)

    def vreg(j, cc):
      sv_ = srcw[pl.ds(_al(j * 16, 16), 16)]
      dv = dstw[pl.ds(_al(j * 16, 16), 16)]
      b = lax.div(dv, BW)
      dloc = dv - b * BW
      word = (sv_ << 10) | dloc
      sk, svl = plsc.sort_key_val(b, i16)
      tag16[...] = sk
      prev = plsc.load_gather(tag16, [jnp.maximum(i16 - 1, 0)])
      nxt = plsc.load_gather(tag16, [jnp.minimum(i16 + 1, 15)])
      first = (i16 == 0) | (sk != prev)
      last = (i16 == 15) | (sk != nxt)
      occ_s = i16 - plsc.cummax(jnp.where(first, i16, 0))
      base_s = plsc.load_gather(cur, [sk])
      pos_s = base_s + occ_s
      plsc.store_scatter(cur, [sk], pos_s + 1, mask=last)
      plsc.store_scatter(pos16, [svl], pos_s)
      wobuf[pl.ds(_al(j * 16, 16), 16)] = word
      pobuf[pl.ds(_al(j * 16, 16), 16)] = pos16[...]
      return cc

    lax.fori_loop(0, WH // 16, vreg, 0)
    pltpu.async_copy(wobuf, words_hbm.at[pobuf], sem).wait()
    return carry2

  lax.fori_loop(0, NWIN_H, win, 0)


def _make_sc_edge(F, FP, with_cnt):
  BWF = BW * F
  n_out = 5 if with_cnt else 4
  outs = [jax.ShapeDtypeStruct((NP * F,), jnp.float32) for _ in range(4)]
  if with_cnt:
    outs.append(jax.ShapeDtypeStruct((NP,), jnp.float32))
  scratch = [
      pltpu.VMEM((2 * WE,), jnp.int32),
      pltpu.VMEM((2 * WE,), jnp.int32),
      pltpu.VMEM((2 * WE, FP), jnp.float32),
      pltpu.VMEM((BWF,), jnp.float32),
      pltpu.VMEM((BWF,), jnp.float32),
      pltpu.VMEM((BWF,), jnp.float32),
      pltpu.VMEM((BWF,), jnp.float32),
      pltpu.VMEM((BW,), jnp.float32),
      pltpu.VMEM((NB,), jnp.int32),
      pltpu.VMEM((NB,), jnp.int32),
      pltpu.VMEM((16,), jnp.int32),
      pltpu.VMEM((16,), jnp.int32),
      pltpu.SemaphoreType.DMA,
      pltpu.SemaphoreType.DMA,
  ]

  @functools.partial(
      pl.kernel, mesh=_mesh(), compiler_params=_SC_PARAMS,
      out_type=outs, scratch_types=scratch)
  def edge(words_hbm, bbase_hbm, bcnt_hbm, btab_hbm, zero_hbm, pinf_hbm,
           ninf_hbm, *rest):
    out_refs = rest[:n_out]
    (wbuf, idxb, brow, sacc, s2acc, mnacc, mxacc, cntacc, mbase, mcnt,
     tag16, occ16, sem_w, sem_g) = rest[n_out:]
    S_hbm, S2_hbm, Mn_hbm, Mx_hbm = out_refs[:4]
    cnt_hbm = out_refs[4] if with_cnt else None

    wid = _wid()
    i16 = _i16()
    onesf = jnp.ones((16,), jnp.float32)
    pltpu.sync_copy(bbase_hbm, mbase)
    pltpu.sync_copy(bcnt_hbm, mcnt)

    def scalar_at(ref, idx):
      chunk = ref[pl.ds(_al(lax.div(idx, 16) * 16, 16), 16)]
      return jnp.sum(jnp.where(i16 == lax.rem(idx, 16), chunk, 0))

    def build_idx(par):
      def bi(j, cc):
        v = wbuf[pl.ds(_al(par * WE + j * 16, 16), 16)]
        idxb[pl.ds(_al(par * WE + j * 16, 16), 16)] = jnp.clip(v >> 10, 0, N - 1)
        return cc

      lax.fori_loop(0, WE // 16, bi, 0)

    def issue_gather(par):
      for k in range(GC):
        pltpu.async_copy(btab_hbm.at[idxb.at[pl.ds(_al(par * WE + k * 128, 16), 128)]],
                         brow.at[pl.ds((par * GC + k) * 128, 128)], sem_g)

    def drain_gather(par):
      for k in range(GC):
        pltpu.make_async_copy(
            btab_hbm.at[pl.ds(0, 128)],
            brow.at[pl.ds((par * GC + k) * 128, 128)], sem_g).wait()

    def bin_body(p, carry0):
      b = wid * PASSES + p
      base = _al(scalar_at(mbase, b))
      cnt = scalar_at(mcnt, b)
      nwin = lax.div(cnt + (WE - 1), WE)
      pltpu.sync_copy(zero_hbm.at[pl.ds(0, BWF)], sacc)
      pltpu.sync_copy(zero_hbm.at[pl.ds(BWF, BWF)], s2acc)
      pltpu.sync_copy(pinf_hbm, mnacc)
      pltpu.sync_copy(ninf_hbm, mxacc)
      if with_cnt:
        pltpu.sync_copy(zero_hbm.at[pl.ds(0, BW)], cntacc)

      @pl.when(nwin > 0)
      def _():
        pltpu.sync_copy(words_hbm.at[pl.ds(base, WE)], wbuf.at[pl.ds(0, WE)])
        build_idx(0)
        issue_gather(0)

      def win_body(w, carry):
        par = lax.rem(w, 2)
        opp = 1 - par

        @pl.when(w + 1 < nwin)
        def _():
          pltpu.async_copy(words_hbm.at[pl.ds(_al(base + (w + 1) * WE), WE)],
                           wbuf.at[pl.ds(_al(opp * WE, 16), WE)], sem_w)

        drain_gather(par)

        def vreg(j, cc2):
          gpos = w * WE + j * 16 + i16
          valid = gpos < cnt
          v = wbuf[pl.ds(_al(par * WE + j * 16, 16), 16)]
          dloc = jnp.minimum(v & 1023, BW - 1)
          rowv = par * WE + j * 16 + i16
          dbase = dloc * F
          if with_cnt:
            plsc.addupdate_scatter(cntacc, [dloc], onesf, mask=valid)
          sk, sv2, om = plsc.sort_key_val(dloc, i16, mask=valid)
          tag16[...] = sk
          prev = plsc.load_gather(tag16, [jnp.maximum(i16 - 1, 0)])
          occ_s = i16 - plsc.cummax(
              jnp.where((i16 == 0) | (sk != prev), i16, 0))
          occ16[...] = jnp.zeros((16,), jnp.int32)
          plsc.store_scatter(occ16, [sv2], occ_s, mask=om)
          occ = occ16[...]
          ncls = jnp.max(jnp.where(om, occ_s, 0))
          for f in range(F):
            fv = jnp.full((16,), f, jnp.int32)
            val = plsc.load_gather(brow, [rowv, fv], mask=valid)
            di = dbase + f
            plsc.addupdate_scatter(sacc, [di], val, mask=valid)
            plsc.addupdate_scatter(s2acc, [di], val * val, mask=valid)

          def cls_body(kcls, cc):
            mk = valid & (occ == kcls)
            for f in range(F):
              fv = jnp.full((16,), f, jnp.int32)
              val = plsc.load_gather(brow, [rowv, fv], mask=mk)
              di = dbase + f
              old = plsc.load_gather(mnacc, [di], mask=mk)
              plsc.store_scatter(mnacc, [di], jnp.minimum(old, val), mask=mk)
              old2 = plsc.load_gather(mxacc, [di], mask=mk)
              plsc.store_scatter(mxacc, [di], jnp.maximum(old2, val), mask=mk)
            return cc

          lax.fori_loop(0, ncls + 1, cls_body, 0)
          return cc2

        lax.fori_loop(0, WE // 16, vreg, 0)

        @pl.when(w + 1 < nwin)
        def _():
          pltpu.make_async_copy(words_hbm.at[pl.ds(0, WE)],
                                wbuf.at[pl.ds(_al(opp * WE, 16), WE)], sem_w).wait()
          build_idx(opp)
          issue_gather(opp)

        return carry

      lax.fori_loop(0, nwin, win_body, 0)
      pltpu.sync_copy(sacc, S_hbm.at[pl.ds(_al(b * BWF), BWF)])
      pltpu.sync_copy(s2acc, S2_hbm.at[pl.ds(_al(b * BWF), BWF)])
      pltpu.sync_copy(mnacc, Mn_hbm.at[pl.ds(_al(b * BWF), BWF)])
      pltpu.sync_copy(mxacc, Mx_hbm.at[pl.ds(_al(b * BWF), BWF)])
      if with_cnt:
        pltpu.sync_copy(cntacc, cnt_hbm.at[pl.ds(_al(b * BW), BW)])
      return carry0

    lax.fori_loop(0, PASSES, bin_body, 0)

  return edge


_sc_edge1 = _make_sc_edge(25, 32, True)
_sc_edge2 = _make_sc_edge(16, 16, False)


_BLK0 = 2000
_BLK = 2048


def _full(spec_shape):
  return pl.BlockSpec(spec_shape, lambda i: (0, 0))


def _tc_pre1_body(x_ref, wsT_ref, b_ref, out_ref):
  bt = jnp.dot(x_ref[...], wsT_ref[...],
               preferred_element_type=jnp.float32) + b_ref[...]
  out_ref[...] = jnp.concatenate(
      [bt, jnp.zeros((bt.shape[0], 7), jnp.float32)], axis=1)


def _tc_pre1(x, Ws1T, bpre1):
  return pl.pallas_call(
      _tc_pre1_body,
      grid=(N // _BLK0,),
      in_specs=[pl.BlockSpec((_BLK0, 25), lambda i: (i, 0)),
                _full((25, 25)), _full((1, 25))],
      out_specs=pl.BlockSpec((_BLK0, 32), lambda i: (i, 0)),
      out_shape=jax.ShapeDtypeStruct((N, 32), jnp.float32),
  )(x, Ws1T, bpre1)


def _pna_combine(xb, A, S, S2, Mn, Mx, c, WxT, W1T, W2T, W3T, bpost):
  cg = c > 0.0
  cm = jnp.maximum(c, 1.0)
  sm = S / cm
  mean = jnp.where(cg, A + sm, 0.0)
  mn = jnp.where(cg, A + Mn, 0.0)
  mx = jnp.where(cg, A + Mx, 0.0)
  std = jnp.sqrt(jax.nn.relu(S2 / cm - sm * sm) + 1e-5)
  agg = jnp.concatenate([mean, mn, mx, std], axis=1)
  logd = jnp.log(cm + 1.0)
  amp = logd * (1.0 / AVG_DEG_LOG)
  att = AVG_DEG_LOG / logd
  out = (jnp.dot(xb, WxT, preferred_element_type=jnp.float32)
         + jnp.dot(agg, W1T, preferred_element_type=jnp.float32)
         + jnp.dot(agg * amp, W2T, preferred_element_type=jnp.float32)
         + jnp.dot(agg * att, W3T, preferred_element_type=jnp.float32)
         + bpost)
  return out


def _tc_post1_body(x_ref, S_ref, S2_ref, Mn_ref, Mx_ref, c_ref,
                   Wd1T_ref, Wx1T_ref, W11T_ref, W21T_ref, W31T_ref,
                   bpost1_ref, Wlin1T_ref, blin1_ref, Ws2T_ref, bpre2_ref,
                   h1_ref, b2tab_ref):
  xb = x_ref[...]
  A = jnp.dot(xb, Wd1T_ref[...], preferred_element_type=jnp.float32)
  out = _pna_combine(xb, A, S_ref[...], S2_ref[...], Mn_ref[...], Mx_ref[...],
                     c_ref[...], Wx1T_ref[...], W11T_ref[...], W21T_ref[...],
                     W31T_ref[...], bpost1_ref[...])
  h = jax.nn.relu(jnp.dot(out, Wlin1T_ref[...],
                          preferred_element_type=jnp.float32) + blin1_ref[...])
  h1_ref[...] = h
  b2tab_ref[...] = jnp.dot(h, Ws2T_ref[...],
                           preferred_element_type=jnp.float32) + bpre2_ref[...]


def _tc_post1(x_pad, S, S2, Mn, Mx, cnt, Wd1T, Wx1T, W11T, W21T, W31T,
              bpost1, Wlin1T, blin1, Ws2T, bpre2):
  nb = pl.BlockSpec((_BLK, 25), lambda i: (i, 0))
  return pl.pallas_call(
      _tc_post1_body,
      grid=(NP // _BLK,),
      in_specs=[nb, nb, nb, nb, nb,
                pl.BlockSpec((_BLK, 1), lambda i: (i, 0)),
                _full((25, 25)), _full((25, 16)), _full((100, 16)),
                _full((100, 16)), _full((100, 16)), _full((1, 16)),
                _full((16, 16)), _full((1, 16)), _full((16, 16)),
                _full((1, 16))],
      out_specs=[pl.BlockSpec((_BLK, 16), lambda i: (i, 0)),
                 pl.BlockSpec((_BLK, 16), lambda i: (i, 0))],
      out_shape=[jax.ShapeDtypeStruct((NP, 16), jnp.float32),
                 jax.ShapeDtypeStruct((NP, 16), jnp.float32)],
  )(x_pad, S, S2, Mn, Mx, cnt, Wd1T, Wx1T, W11T, W21T, W31T, bpost1,
    Wlin1T, blin1, Ws2T, bpre2)


def _tc_post2_body(h1_ref, S_ref, S2_ref, Mn_ref, Mx_ref, c_ref, batch_ref,
                   Wd2T_ref, Wx2T_ref, W12T_ref, W22T_ref, W32T_ref,
                   bpost2_ref, Wlin2T_ref, blin2_ref, WfcT_ref, bfc_ref,
                   out_ref, acc_ref):
  i = pl.program_id(0)
  hb = h1_ref[...]
  A = jnp.dot(hb, Wd2T_ref[...], preferred_element_type=jnp.float32)
  out = _pna_combine(hb, A, S_ref[...], S2_ref[...], Mn_ref[...], Mx_ref[...],
                     c_ref[...], Wx2T_ref[...], W12T_ref[...], W22T_ref[...],
                     W32T_ref[...], bpost2_ref[...])
  h2 = jax.nn.relu(jnp.dot(out, Wlin2T_ref[...],
                           preferred_element_type=jnp.float32) + blin2_ref[...])
  rows = i * _BLK + lax.broadcasted_iota(jnp.int32, (_BLK, 1), 0)
  rmask = rows < N
  h2 = jnp.where(rmask, h2, 0.0)
  onehot = ((batch_ref[...] == lax.broadcasted_iota(jnp.int32, (_BLK, G), 1))
            & rmask).astype(jnp.float32)
  contrib = lax.dot_general(onehot, h2, (((0,), (0,)), ((), ())),
                            preferred_element_type=jnp.float32)

  @pl.when(i == 0)
  def _():
    acc_ref[...] = jnp.zeros_like(acc_ref)

  acc_ref[...] += contrib

  @pl.when(i == NP // _BLK - 1)
  def _():
    logits = jnp.dot(acc_ref[...], WfcT_ref[...],
                     preferred_element_type=jnp.float32) + bfc_ref[...]
    m = jnp.max(logits, axis=1, keepdims=True)
    ex = jnp.exp(logits - m)
    out_ref[...] = (logits - m) - jnp.log(jnp.sum(ex, axis=1, keepdims=True))


def _tc_post2(h1, S, S2, Mn, Mx, cnt, batch_pad, Wd2T, Wx2T, W12T, W22T, W32T,
              bpost2, Wlin2T, blin2, WfcT, bfc):
  nb = pl.BlockSpec((_BLK, 16), lambda i: (i, 0))
  cb = pl.BlockSpec((_BLK, 1), lambda i: (i, 0))
  return pl.pallas_call(
      _tc_post2_body,
      grid=(NP // _BLK,),
      in_specs=[nb, nb, nb, nb, nb, cb, cb,
                _full((16, 16)), _full((16, 8)), _full((64, 8)),
                _full((64, 8)), _full((64, 8)), _full((1, 8)),
                _full((8, 8)), _full((1, 8)), _full((8, 2)), _full((1, 2))],
      out_specs=pl.BlockSpec((G, 2), lambda i: (0, 0)),
      out_shape=jax.ShapeDtypeStruct((G, 2), jnp.float32),
      scratch_shapes=[pltpu.VMEM((G, 8), jnp.float32)],
  )(h1, S, S2, Mn, Mx, cnt, batch_pad, Wd2T, Wx2T, W12T, W22T, W32T, bpost2,
    Wlin2T, blin2, WfcT, bfc)


def kernel(x, edge_index, batch,
           Wpre1, bpre1, Wpost1, bpost1, Wlin1, blin1,
           Wpre2, bpre2, Wpost2, bpost2, Wlin2, blin2,
           Wfc, bfc):
  src = edge_index[0]
  dst = edge_index[1]

  hist = _sc_hist(dst)
  bwords, bbase, bcnt = _sc_permute(src, dst, hist)

  Wd1T = Wpre1[:, :25].T
  Ws1T = Wpre1[:, 25:].T
  Wx1T = Wpost1[:, :25].T
  W11T = Wpost1[:, 25:125].T
  W21T = Wpost1[:, 125:225].T
  W31T = Wpost1[:, 225:325].T
  Wd2T = Wpre2[:, :16].T
  Ws2T = Wpre2[:, 16:].T
  Wx2T = Wpost2[:, :16].T
  W12T = Wpost2[:, 16:80].T
  W22T = Wpost2[:, 80:144].T
  W32T = Wpost2[:, 144:208].T

  BWF1 = BW * 25
  zero_t = jnp.zeros((2 * BWF1,), jnp.float32)
  pinf_t = jnp.full((BWF1,), jnp.inf, jnp.float32)
  ninf_t = jnp.full((BWF1,), -jnp.inf, jnp.float32)
  zero_t2 = jnp.zeros((2 * BW * 16,), jnp.float32)
  pinf_t2 = jnp.full((BW * 16,), jnp.inf, jnp.float32)
  ninf_t2 = jnp.full((BW * 16,), -jnp.inf, jnp.float32)

  b1tab = _tc_pre1(x, Ws1T, bpre1.reshape(1, 25))
  S1, S21, Mn1, Mx1, cntf = _sc_edge1(bwords, bbase, bcnt, b1tab,
                                      zero_t, pinf_t, ninf_t)
  x_pad = jnp.pad(x, ((0, NP - N), (0, 0)))
  cnt2d = cntf.reshape(NP, 1)
  h1, b2tab = _tc_post1(
      x_pad, S1.reshape(NP, 25), S21.reshape(NP, 25), Mn1.reshape(NP, 25),
      Mx1.reshape(NP, 25), cnt2d, Wd1T, Wx1T, W11T, W21T, W31T,
      bpost1.reshape(1, 16), Wlin1.T, blin1.reshape(1, 16), Ws2T,
      bpre2.reshape(1, 16))

  S2_, S22, Mn2, Mx2 = _sc_edge2(bwords, bbase, bcnt, b2tab,
                                 zero_t2, pinf_t2, ninf_t2)
  batch_pad = jnp.pad(batch, (0, NP - N)).reshape(NP, 1)
  out = _tc_post2(
      h1, S2_.reshape(NP, 16), S22.reshape(NP, 16), Mn2.reshape(NP, 16),
      Mx2.reshape(NP, 16), cnt2d, batch_pad, Wd2T, Wx2T, W12T, W22T, W32T,
      bpost2.reshape(1, 8), Wlin2.T, blin2.reshape(1, 8), Wfc.T,
      bfc.reshape(1, 2))
  return out


# speedup vs baseline: 5.7029x; 1.0603x over previous
import functools
import numpy as np
import jax
import jax.numpy as jnp
from jax import lax
from jax.experimental import pallas as pl
from jax.experimental.pallas import tpu as pltpu
from jax.experimental.pallas import tpu_sc as plsc

N = 100000
E = 1600000
G = 128
AVG_DEG_LOG = float(np.log(17.0))

NC = 2
NS = 16
NW = NC * NS
BW = 784
NB = 128
NP = BW * NB
PASSES = NB // NW

EW = E // NW
WH = 2000
NWIN_H = EW // WH

EP = 1602048
TRASH = EP - 64

WE = 512
GC = WE // 128

_SC_PARAMS = pltpu.CompilerParams(
    needs_layout_passes=False, use_tc_tiling_on_sc=False)


def _mesh():
  return plsc.VectorSubcoreMesh(core_axis_name="c", subcore_axis_name="s")


def _wid():
  return lax.axis_index("s") * NC + lax.axis_index("c")


def _i16():
  return lax.iota(jnp.int32, 16)


def _al(v, m=8):
  return pl.multiple_of(v, m)


@functools.partial(
    pl.kernel, mesh=_mesh(), compiler_params=_SC_PARAMS,
    out_type=jax.ShapeDtypeStruct((NW * NB,), jnp.int32),
    scratch_types=[pltpu.VMEM((2 * WH,), jnp.int32),
                   pltpu.VMEM((NB,), jnp.int32),
                   pltpu.SemaphoreType.DMA],
)
def _sc_hist(dst_hbm, hist_hbm, dstw, histv, sem):
  wid = _wid()
  for j in range(NB // 16):
    histv[pl.ds(j * 16, 16)] = jnp.zeros((16,), jnp.int32)
  ones = jnp.ones((16,), jnp.int32)
  pltpu.sync_copy(dst_hbm.at[pl.ds(_al(wid * EW), WH)], dstw.at[pl.ds(0, WH)])

  def win(w, carry):
    par = lax.rem(w, 2)
    opp = 1 - par

    @pl.when(w + 1 < NWIN_H)
    def _():
      pltpu.async_copy(dst_hbm.at[pl.ds(_al(wid * EW + (w + 1) * WH), WH)],
                       dstw.at[pl.ds(_al(opp * WH, 16), WH)], sem)

    def vreg(j, cc):
      v = dstw[pl.ds(_al(par * WH + j * 16, 16), 16)]
      b = lax.div(v, BW)
      plsc.addupdate_scatter(histv, [b], ones)
      return cc

    lax.fori_loop(0, WH // 16, vreg, 0)

    @pl.when(w + 1 < NWIN_H)
    def _():
      pltpu.make_async_copy(dst_hbm.at[pl.ds(0, WH)],
                            dstw.at[pl.ds(_al(opp * WH, 16), WH)], sem).wait()
    return carry

  lax.fori_loop(0, NWIN_H, win, 0)
  pltpu.sync_copy(histv, hist_hbm.at[pl.ds(_al(wid * NB), NB)])


@functools.partial(
    pl.kernel, mesh=_mesh(), compiler_params=_SC_PARAMS,
    out_type=[jax.ShapeDtypeStruct((EP,), jnp.int32),
              jax.ShapeDtypeStruct((NB,), jnp.int32),
              jax.ShapeDtypeStruct((NB,), jnp.int32)],
    scratch_types=[pltpu.VMEM((2 * WH,), jnp.int32),
                   pltpu.VMEM((2 * WH,), jnp.int32),
                   pltpu.VMEM((NW * NB,), jnp.int32),
                   pltpu.VMEM((NB,), jnp.int32),
                   pltpu.VMEM((NB,), jnp.int32),
                   pltpu.VMEM((NB,), jnp.int32),
                   pltpu.VMEM((2 * WH,), jnp.int32),
                   pltpu.VMEM((2, WH), jnp.int32),
                   pltpu.VMEM((16,), jnp.int32),
                   pltpu.VMEM((16,), jnp.int32),
                   pltpu.SemaphoreType.DMA,
                   pltpu.SemaphoreType.DMA],
)
def _sc_permute(src_hbm, dst_hbm, hist_hbm, words_hbm, bbase_hbm, bcnt_hbm,
                srcw, dstw, histall, cntv, basev, cur, wobuf, pobuf,
                tag16, pos16, sem, sem_in):
  wid = _wid()
  i16 = _i16()
  pltpu.sync_copy(hist_hbm, histall)

  carry = jnp.int32(0)
  for bb in range(NB // 16):
    tot = jnp.zeros((16,), jnp.int32)
    for w in range(NW):
      tot = tot + histall[pl.ds(_al(w * NB + bb * 16, 16), 16)]
    cntv[pl.ds(bb * 16, 16)] = tot
    a8 = (tot + 7) & ~7
    incl = plsc.cumsum(a8)
    basev[pl.ds(bb * 16, 16)] = carry + incl - a8
    carry = carry + jnp.sum(a8)

    def add_prev(w, s16):
      return s16 + histall[pl.ds(_al(w * NB + bb * 16, 16), 16)]

    mine = lax.fori_loop(0, wid, add_prev, basev[pl.ds(bb * 16, 16)])
    cur[pl.ds(bb * 16, 16)] = mine

  @pl.when(wid == 0)
  def _():
    pltpu.sync_copy(basev, bbase_hbm)
    pltpu.sync_copy(cntv, bcnt_hbm)

  pltpu.sync_copy(src_hbm.at[pl.ds(_al(wid * EW), WH)], srcw.at[pl.ds(0, WH)])
  pltpu.sync_copy(dst_hbm.at[pl.ds(_al(wid * EW), WH)], dstw.at[pl.ds(0, WH)])

  def win(w, carry2):
    par = lax.rem(w, 2)
    opp = 1 - par

    @pl.when(w + 1 < NWIN_H)
    def _():
      off = _al(wid * EW + (w + 1) * WH)
      pltpu.async_copy(src_hbm.at[pl.ds(off, WH)],
                       srcw.at[pl.ds(_al(opp * WH, 16), WH)], sem_in)
      pltpu.async_copy(dst_hbm.at[pl.ds(off, WH)],
                       dstw.at[pl.ds(_al(opp * WH, 16), WH)], sem_in)

    def vreg(j, cc):
      sv_ = srcw[pl.ds(_al(par * WH + j * 16, 16), 16)]
      dv = dstw[pl.ds(_al(par * WH + j * 16, 16), 16)]
      b = lax.div(dv, BW)
      dloc = dv - b * BW
      word = (sv_ << 10) | dloc
      sk, svl = plsc.sort_key_val(b, i16)
      tag16[...] = sk
      prev = plsc.load_gather(tag16, [jnp.maximum(i16 - 1, 0)])
      nxt = plsc.load_gather(tag16, [jnp.minimum(i16 + 1, 15)])
      first = (i16 == 0) | (sk != prev)
      last = (i16 == 15) | (sk != nxt)
      occ_s = i16 - plsc.cummax(jnp.where(first, i16, 0))
      base_s = plsc.load_gather(cur, [sk])
      pos_s = base_s + occ_s
      plsc.store_scatter(cur, [sk], pos_s + 1, mask=last)
      plsc.store_scatter(pos16, [svl], pos_s)
      wobuf[pl.ds(_al(par * WH + j * 16, 16), 16)] = word
      pobuf[par, pl.ds(_al(j * 16, 16), 16)] = pos16[...]
      return cc

    lax.fori_loop(0, WH // 16, vreg, 0)

    @pl.when(w > 0)
    def _():
      pltpu.make_async_copy(src_hbm.at[pl.ds(0, WH)],
                            wobuf.at[pl.ds(_al(opp * WH, 16), WH)], sem).wait()
    pltpu.async_copy(wobuf.at[pl.ds(_al(par * WH, 16), WH)],
                     words_hbm.at[pobuf.at[par]], sem)

    @pl.when(w + 1 < NWIN_H)
    def _():
      pltpu.make_async_copy(src_hbm.at[pl.ds(0, WH)],
                            srcw.at[pl.ds(_al(opp * WH, 16), WH)],
                            sem_in).wait()
      pltpu.make_async_copy(src_hbm.at[pl.ds(0, WH)],
                            dstw.at[pl.ds(_al(opp * WH, 16), WH)],
                            sem_in).wait()
    return carry2

  lax.fori_loop(0, NWIN_H, win, 0)
  pltpu.make_async_copy(src_hbm.at[pl.ds(0, WH)],
                        wobuf.at[pl.ds(((NWIN_H - 1) % 2) * WH, WH)],
                        sem).wait()


def _make_sc_edge(F, FP, with_cnt):
  BWF = BW * F
  n_out = 5 if with_cnt else 4
  outs = [jax.ShapeDtypeStruct((NP * F,), jnp.float32) for _ in range(4)]
  if with_cnt:
    outs.append(jax.ShapeDtypeStruct((NP,), jnp.float32))
  scratch = [
      pltpu.VMEM((2 * WE,), jnp.int32),
      pltpu.VMEM((2 * WE,), jnp.int32),
      pltpu.VMEM((2 * WE, FP), jnp.float32),
      pltpu.VMEM((BWF,), jnp.float32),
      pltpu.VMEM((BWF,), jnp.float32),
      pltpu.VMEM((BWF,), jnp.float32),
      pltpu.VMEM((BWF,), jnp.float32),
      pltpu.VMEM((BW,), jnp.float32),
      pltpu.VMEM((NB,), jnp.int32),
      pltpu.VMEM((NB,), jnp.int32),
      pltpu.VMEM((16,), jnp.int32),
      pltpu.VMEM((16,), jnp.int32),
      pltpu.SemaphoreType.DMA,
      pltpu.SemaphoreType.DMA,
  ]

  @functools.partial(
      pl.kernel, mesh=_mesh(), compiler_params=_SC_PARAMS,
      out_type=outs, scratch_types=scratch)
  def edge(words_hbm, bbase_hbm, bcnt_hbm, btab_hbm, zero_hbm, pinf_hbm,
           ninf_hbm, *rest):
    out_refs = rest[:n_out]
    (wbuf, idxb, brow, sacc, s2acc, mnacc, mxacc, cntacc, mbase, mcnt,
     tag16, occ16, sem_w, sem_g) = rest[n_out:]
    S_hbm, S2_hbm, Mn_hbm, Mx_hbm = out_refs[:4]
    cnt_hbm = out_refs[4] if with_cnt else None

    wid = _wid()
    i16 = _i16()
    onesf = jnp.ones((16,), jnp.float32)
    pltpu.sync_copy(bbase_hbm, mbase)
    pltpu.sync_copy(bcnt_hbm, mcnt)

    def scalar_at(ref, idx):
      chunk = ref[pl.ds(_al(lax.div(idx, 16) * 16, 16), 16)]
      return jnp.sum(jnp.where(i16 == lax.rem(idx, 16), chunk, 0))

    def build_idx(par):
      def bi(j, cc):
        v = wbuf[pl.ds(_al(par * WE + j * 16, 16), 16)]
        idxb[pl.ds(_al(par * WE + j * 16, 16), 16)] = jnp.clip(v >> 10, 0, N - 1)
        return cc

      lax.fori_loop(0, WE // 16, bi, 0)

    def issue_gather(par):
      for k in range(GC):
        pltpu.async_copy(btab_hbm.at[idxb.at[pl.ds(_al(par * WE + k * 128, 16), 128)]],
                         brow.at[pl.ds((par * GC + k) * 128, 128)], sem_g)

    def drain_gather(par):
      for k in range(GC):
        pltpu.make_async_copy(
            btab_hbm.at[pl.ds(0, 128)],
            brow.at[pl.ds((par * GC + k) * 128, 128)], sem_g).wait()

    def bin_body(p, carry0):
      b = wid * PASSES + p
      base = _al(scalar_at(mbase, b))
      cnt = scalar_at(mcnt, b)
      nwin = lax.div(cnt + (WE - 1), WE)
      pltpu.sync_copy(zero_hbm.at[pl.ds(0, BWF)], sacc)
      pltpu.sync_copy(zero_hbm.at[pl.ds(BWF, BWF)], s2acc)
      pltpu.sync_copy(pinf_hbm, mnacc)
      pltpu.sync_copy(ninf_hbm, mxacc)
      if with_cnt:
        pltpu.sync_copy(zero_hbm.at[pl.ds(0, BW)], cntacc)

      @pl.when(nwin > 0)
      def _():
        pltpu.sync_copy(words_hbm.at[pl.ds(base, WE)], wbuf.at[pl.ds(0, WE)])
        build_idx(0)
        issue_gather(0)

      @pl.when(nwin > 1)
      def _():
        pltpu.async_copy(words_hbm.at[pl.ds(_al(base + WE), WE)],
                         wbuf.at[pl.ds(WE, WE)], sem_w)

      def win_body(w, carry):
        par = lax.rem(w, 2)
        opp = 1 - par

        @pl.when(w + 1 < nwin)
        def _():
          pltpu.make_async_copy(words_hbm.at[pl.ds(0, WE)],
                                wbuf.at[pl.ds(_al(opp * WE, 16), WE)],
                                sem_w).wait()
          build_idx(opp)
          issue_gather(opp)

        drain_gather(par)

        def vreg(j, cc2):
          gpos = w * WE + j * 16 + i16
          valid = gpos < cnt
          v = wbuf[pl.ds(_al(par * WE + j * 16, 16), 16)]
          dloc = jnp.minimum(v & 1023, BW - 1)
          rowv = par * WE + j * 16 + i16
          dbase = dloc * F
          if with_cnt:
            plsc.addupdate_scatter(cntacc, [dloc], onesf, mask=valid)
          sk, sv2, om = plsc.sort_key_val(dloc, i16, mask=valid)
          tag16[...] = sk
          prev = plsc.load_gather(tag16, [jnp.maximum(i16 - 1, 0)])
          occ_s = i16 - plsc.cummax(
              jnp.where((i16 == 0) | (sk != prev), i16, 0))
          occ16[...] = jnp.zeros((16,), jnp.int32)
          plsc.store_scatter(occ16, [sv2], occ_s, mask=om)
          occ = occ16[...]
          ncls = jnp.max(jnp.where(om, occ_s, 0))
          for f in range(F):
            fv = jnp.full((16,), f, jnp.int32)
            val = plsc.load_gather(brow, [rowv, fv], mask=valid)
            di = dbase + f
            plsc.addupdate_scatter(sacc, [di], val, mask=valid)
            plsc.addupdate_scatter(s2acc, [di], val * val, mask=valid)

          def cls_body(kcls, cc):
            mk = valid & (occ == kcls)
            for f in range(F):
              fv = jnp.full((16,), f, jnp.int32)
              val = plsc.load_gather(brow, [rowv, fv], mask=mk)
              di = dbase + f
              old = plsc.load_gather(mnacc, [di], mask=mk)
              plsc.store_scatter(mnacc, [di], jnp.minimum(old, val), mask=mk)
              old2 = plsc.load_gather(mxacc, [di], mask=mk)
              plsc.store_scatter(mxacc, [di], jnp.maximum(old2, val), mask=mk)
            return cc

          lax.fori_loop(0, ncls + 1, cls_body, 0)
          return cc2

        lax.fori_loop(0, WE // 16, vreg, 0)

        @pl.when(w + 2 < nwin)
        def _():
          pltpu.async_copy(words_hbm.at[pl.ds(_al(base + (w + 2) * WE), WE)],
                           wbuf.at[pl.ds(_al(par * WE, 16), WE)], sem_w)

        return carry

      lax.fori_loop(0, nwin, win_body, 0)
      pltpu.sync_copy(sacc, S_hbm.at[pl.ds(_al(b * BWF), BWF)])
      pltpu.sync_copy(s2acc, S2_hbm.at[pl.ds(_al(b * BWF), BWF)])
      pltpu.sync_copy(mnacc, Mn_hbm.at[pl.ds(_al(b * BWF), BWF)])
      pltpu.sync_copy(mxacc, Mx_hbm.at[pl.ds(_al(b * BWF), BWF)])
      if with_cnt:
        pltpu.sync_copy(cntacc, cnt_hbm.at[pl.ds(_al(b * BW), BW)])
      return carry0

    lax.fori_loop(0, PASSES, bin_body, 0)

  return edge


_sc_edge1 = _make_sc_edge(25, 32, True)
_sc_edge2 = _make_sc_edge(16, 16, False)


_BLK0 = 2000
_BLK = 2048


def _full(spec_shape):
  return pl.BlockSpec(spec_shape, lambda i: (0, 0))


def _tc_pre1_body(x_ref, wsT_ref, b_ref, out_ref):
  bt = jnp.dot(x_ref[...], wsT_ref[...],
               preferred_element_type=jnp.float32) + b_ref[...]
  out_ref[...] = jnp.concatenate(
      [bt, jnp.zeros((bt.shape[0], 7), jnp.float32)], axis=1)


def _tc_pre1(x, Ws1T, bpre1):
  return pl.pallas_call(
      _tc_pre1_body,
      grid=(N // _BLK0,),
      in_specs=[pl.BlockSpec((_BLK0, 25), lambda i: (i, 0)),
                _full((25, 25)), _full((1, 25))],
      out_specs=pl.BlockSpec((_BLK0, 32), lambda i: (i, 0)),
      out_shape=jax.ShapeDtypeStruct((N, 32), jnp.float32),
  )(x, Ws1T, bpre1)


def _pna_combine(xb, A, S, S2, Mn, Mx, c, WxT, W1T, W2T, W3T, bpost):
  cg = c > 0.0
  cm = jnp.maximum(c, 1.0)
  sm = S / cm
  mean = jnp.where(cg, A + sm, 0.0)
  mn = jnp.where(cg, A + Mn, 0.0)
  mx = jnp.where(cg, A + Mx, 0.0)
  std = jnp.sqrt(jax.nn.relu(S2 / cm - sm * sm) + 1e-5)
  agg = jnp.concatenate([mean, mn, mx, std], axis=1)
  logd = jnp.log(cm + 1.0)
  amp = logd * (1.0 / AVG_DEG_LOG)
  att = AVG_DEG_LOG / logd
  out = (jnp.dot(xb, WxT, preferred_element_type=jnp.float32)
         + jnp.dot(agg, W1T, preferred_element_type=jnp.float32)
         + jnp.dot(agg * amp, W2T, preferred_element_type=jnp.float32)
         + jnp.dot(agg * att, W3T, preferred_element_type=jnp.float32)
         + bpost)
  return out


def _tc_post1_body(x_ref, S_ref, S2_ref, Mn_ref, Mx_ref, c_ref,
                   Wd1T_ref, Wx1T_ref, W11T_ref, W21T_ref, W31T_ref,
                   bpost1_ref, Wlin1T_ref, blin1_ref, Ws2T_ref, bpre2_ref,
                   h1_ref, b2tab_ref):
  xb = x_ref[...]
  A = jnp.dot(xb, Wd1T_ref[...], preferred_element_type=jnp.float32)
  out = _pna_combine(xb, A, S_ref[...], S2_ref[...], Mn_ref[...], Mx_ref[...],
                     c_ref[...], Wx1T_ref[...], W11T_ref[...], W21T_ref[...],
                     W31T_ref[...], bpost1_ref[...])
  h = jax.nn.relu(jnp.dot(out, Wlin1T_ref[...],
                          preferred_element_type=jnp.float32) + blin1_ref[...])
  h1_ref[...] = h
  b2tab_ref[...] = jnp.dot(h, Ws2T_ref[...],
                           preferred_element_type=jnp.float32) + bpre2_ref[...]


def _tc_post1(x_pad, S, S2, Mn, Mx, cnt, Wd1T, Wx1T, W11T, W21T, W31T,
              bpost1, Wlin1T, blin1, Ws2T, bpre2):
  nb = pl.BlockSpec((_BLK, 25), lambda i: (i, 0))
  return pl.pallas_call(
      _tc_post1_body,
      grid=(NP // _BLK,),
      in_specs=[nb, nb, nb, nb, nb,
                pl.BlockSpec((_BLK, 1), lambda i: (i, 0)),
                _full((25, 25)), _full((25, 16)), _full((100, 16)),
                _full((100, 16)), _full((100, 16)), _full((1, 16)),
                _full((16, 16)), _full((1, 16)), _full((16, 16)),
                _full((1, 16))],
      out_specs=[pl.BlockSpec((_BLK, 16), lambda i: (i, 0)),
                 pl.BlockSpec((_BLK, 16), lambda i: (i, 0))],
      out_shape=[jax.ShapeDtypeStruct((NP, 16), jnp.float32),
                 jax.ShapeDtypeStruct((NP, 16), jnp.float32)],
  )(x_pad, S, S2, Mn, Mx, cnt, Wd1T, Wx1T, W11T, W21T, W31T, bpost1,
    Wlin1T, blin1, Ws2T, bpre2)


def _tc_post2_body(h1_ref, S_ref, S2_ref, Mn_ref, Mx_ref, c_ref, batch_ref,
                   Wd2T_ref, Wx2T_ref, W12T_ref, W22T_ref, W32T_ref,
                   bpost2_ref, Wlin2T_ref, blin2_ref, WfcT_ref, bfc_ref,
                   out_ref, acc_ref):
  i = pl.program_id(0)
  hb = h1_ref[...]
  A = jnp.dot(hb, Wd2T_ref[...], preferred_element_type=jnp.float32)
  out = _pna_combine(hb, A, S_ref[...], S2_ref[...], Mn_ref[...], Mx_ref[...],
                     c_ref[...], Wx2T_ref[...], W12T_ref[...], W22T_ref[...],
                     W32T_ref[...], bpost2_ref[...])
  h2 = jax.nn.relu(jnp.dot(out, Wlin2T_ref[...],
                           preferred_element_type=jnp.float32) + blin2_ref[...])
  rows = i * _BLK + lax.broadcasted_iota(jnp.int32, (_BLK, 1), 0)
  rmask = rows < N
  h2 = jnp.where(rmask, h2, 0.0)
  onehot = ((batch_ref[...] == lax.broadcasted_iota(jnp.int32, (_BLK, G), 1))
            & rmask).astype(jnp.float32)
  contrib = lax.dot_general(onehot, h2, (((0,), (0,)), ((), ())),
                            preferred_element_type=jnp.float32)

  @pl.when(i == 0)
  def _():
    acc_ref[...] = jnp.zeros_like(acc_ref)

  acc_ref[...] += contrib

  @pl.when(i == NP // _BLK - 1)
  def _():
    logits = jnp.dot(acc_ref[...], WfcT_ref[...],
                     preferred_element_type=jnp.float32) + bfc_ref[...]
    m = jnp.max(logits, axis=1, keepdims=True)
    ex = jnp.exp(logits - m)
    out_ref[...] = (logits - m) - jnp.log(jnp.sum(ex, axis=1, keepdims=True))


def _tc_post2(h1, S, S2, Mn, Mx, cnt, batch_pad, Wd2T, Wx2T, W12T, W22T, W32T,
              bpost2, Wlin2T, blin2, WfcT, bfc):
  nb = pl.BlockSpec((_BLK, 16), lambda i: (i, 0))
  cb = pl.BlockSpec((_BLK, 1), lambda i: (i, 0))
  return pl.pallas_call(
      _tc_post2_body,
      grid=(NP // _BLK,),
      in_specs=[nb, nb, nb, nb, nb, cb, cb,
                _full((16, 16)), _full((16, 8)), _full((64, 8)),
                _full((64, 8)), _full((64, 8)), _full((1, 8)),
                _full((8, 8)), _full((1, 8)), _full((8, 2)), _full((1, 2))],
      out_specs=pl.BlockSpec((G, 2), lambda i: (0, 0)),
      out_shape=jax.ShapeDtypeStruct((G, 2), jnp.float32),
      scratch_shapes=[pltpu.VMEM((G, 8), jnp.float32)],
  )(h1, S, S2, Mn, Mx, cnt, batch_pad, Wd2T, Wx2T, W12T, W22T, W32T, bpost2,
    Wlin2T, blin2, WfcT, bfc)


def kernel(x, edge_index, batch,
           Wpre1, bpre1, Wpost1, bpost1, Wlin1, blin1,
           Wpre2, bpre2, Wpost2, bpost2, Wlin2, blin2,
           Wfc, bfc):
  src = edge_index[0]
  dst = edge_index[1]

  hist = _sc_hist(dst)
  bwords, bbase, bcnt = _sc_permute(src, dst, hist)

  Wd1T = Wpre1[:, :25].T
  Ws1T = Wpre1[:, 25:].T
  Wx1T = Wpost1[:, :25].T
  W11T = Wpost1[:, 25:125].T
  W21T = Wpost1[:, 125:225].T
  W31T = Wpost1[:, 225:325].T
  Wd2T = Wpre2[:, :16].T
  Ws2T = Wpre2[:, 16:].T
  Wx2T = Wpost2[:, :16].T
  W12T = Wpost2[:, 16:80].T
  W22T = Wpost2[:, 80:144].T
  W32T = Wpost2[:, 144:208].T

  BWF1 = BW * 25
  zero_t = jnp.zeros((2 * BWF1,), jnp.float32)
  pinf_t = jnp.full((BWF1,), jnp.inf, jnp.float32)
  ninf_t = jnp.full((BWF1,), -jnp.inf, jnp.float32)
  zero_t2 = jnp.zeros((2 * BW * 16,), jnp.float32)
  pinf_t2 = jnp.full((BW * 16,), jnp.inf, jnp.float32)
  ninf_t2 = jnp.full((BW * 16,), -jnp.inf, jnp.float32)

  b1tab = _tc_pre1(x, Ws1T, bpre1.reshape(1, 25))
  S1, S21, Mn1, Mx1, cntf = _sc_edge1(bwords, bbase, bcnt, b1tab,
                                      zero_t, pinf_t, ninf_t)
  x_pad = jnp.pad(x, ((0, NP - N), (0, 0)))
  cnt2d = cntf.reshape(NP, 1)
  h1, b2tab = _tc_post1(
      x_pad, S1.reshape(NP, 25), S21.reshape(NP, 25), Mn1.reshape(NP, 25),
      Mx1.reshape(NP, 25), cnt2d, Wd1T, Wx1T, W11T, W21T, W31T,
      bpost1.reshape(1, 16), Wlin1.T, blin1.reshape(1, 16), Ws2T,
      bpre2.reshape(1, 16))

  S2_, S22, Mn2, Mx2 = _sc_edge2(bwords, bbase, bcnt, b2tab,
                                 zero_t2, pinf_t2, ninf_t2)
  batch_pad = jnp.pad(batch, (0, NP - N)).reshape(NP, 1)
  out = _tc_post2(
      h1, S2_.reshape(NP, 16), S22.reshape(NP, 16), Mn2.reshape(NP, 16),
      Mx2.reshape(NP, 16), cnt2d, batch_pad, Wd2T, Wx2T, W12T, W22T, W32T,
      bpost2.reshape(1, 8), Wlin2.T, blin2.reshape(1, 8), Wfc.T,
      bfc.reshape(1, 2))
  return out


# speedup vs baseline: 6.3797x; 1.1187x over previous
import functools
import numpy as np
import jax
import jax.numpy as jnp
from jax import lax
from jax.experimental import pallas as pl
from jax.experimental.pallas import tpu as pltpu
from jax.experimental.pallas import tpu_sc as plsc

N = 100000
E = 1600000
G = 128
AVG_DEG_LOG = float(np.log(17.0))

NC = 2
NS = 16
NW = NC * NS
BW = 784
NB = 128
NP = BW * NB
PASSES = NB // NW

EW = E // NW
WH = 2000
NWIN_H = EW // WH

EP = 1602048
TRASH = EP - 64

WE = 512
GC = WE // 128

_SC_PARAMS = pltpu.CompilerParams(
    needs_layout_passes=False, use_tc_tiling_on_sc=False)


def _mesh():
  return plsc.VectorSubcoreMesh(core_axis_name="c", subcore_axis_name="s")


def _wid():
  return lax.axis_index("s") * NC + lax.axis_index("c")


def _i16():
  return lax.iota(jnp.int32, 16)


def _al(v, m=8):
  return pl.multiple_of(v, m)


@functools.partial(
    pl.kernel, mesh=_mesh(), compiler_params=_SC_PARAMS,
    out_type=jax.ShapeDtypeStruct((NW * NB,), jnp.int32),
    scratch_types=[pltpu.VMEM((2 * WH,), jnp.int32),
                   pltpu.VMEM((NB,), jnp.int32),
                   pltpu.SemaphoreType.DMA],
)
def _sc_hist(dst_hbm, hist_hbm, dstw, histv, sem):
  wid = _wid()
  for j in range(NB // 16):
    histv[pl.ds(j * 16, 16)] = jnp.zeros((16,), jnp.int32)
  ones = jnp.ones((16,), jnp.int32)
  pltpu.sync_copy(dst_hbm.at[pl.ds(_al(wid * EW), WH)], dstw.at[pl.ds(0, WH)])

  def win(w, carry):
    par = lax.rem(w, 2)
    opp = 1 - par

    @pl.when(w + 1 < NWIN_H)
    def _():
      pltpu.async_copy(dst_hbm.at[pl.ds(_al(wid * EW + (w + 1) * WH), WH)],
                       dstw.at[pl.ds(_al(opp * WH, 16), WH)], sem)

    def vreg(j, cc):
      v = dstw[pl.ds(_al(par * WH + j * 16, 16), 16)]
      b = lax.div(v, BW)
      plsc.addupdate_scatter(histv, [b], ones)
      return cc

    lax.fori_loop(0, WH // 16, vreg, 0)

    @pl.when(w + 1 < NWIN_H)
    def _():
      pltpu.make_async_copy(dst_hbm.at[pl.ds(0, WH)],
                            dstw.at[pl.ds(_al(opp * WH, 16), WH)], sem).wait()
    return carry

  lax.fori_loop(0, NWIN_H, win, 0)
  pltpu.sync_copy(histv, hist_hbm.at[pl.ds(_al(wid * NB), NB)])


@functools.partial(
    pl.kernel, mesh=_mesh(), compiler_params=_SC_PARAMS,
    out_type=[jax.ShapeDtypeStruct((EP,), jnp.int32),
              jax.ShapeDtypeStruct((NB,), jnp.int32),
              jax.ShapeDtypeStruct((NB,), jnp.int32)],
    scratch_types=[pltpu.VMEM((2 * WH,), jnp.int32),
                   pltpu.VMEM((2 * WH,), jnp.int32),
                   pltpu.VMEM((NW * NB,), jnp.int32),
                   pltpu.VMEM((NB,), jnp.int32),
                   pltpu.VMEM((NB,), jnp.int32),
                   pltpu.VMEM((NB,), jnp.int32),
                   pltpu.VMEM((2 * WH,), jnp.int32),
                   pltpu.VMEM((2, WH), jnp.int32),
                   pltpu.VMEM((16,), jnp.int32),
                   pltpu.VMEM((16,), jnp.int32),
                   pltpu.SemaphoreType.DMA,
                   pltpu.SemaphoreType.DMA],
)
def _sc_permute(src_hbm, dst_hbm, hist_hbm, words_hbm, bbase_hbm, bcnt_hbm,
                srcw, dstw, histall, cntv, basev, cur, wobuf, pobuf,
                tag16, pos16, sem, sem_in):
  wid = _wid()
  i16 = _i16()
  pltpu.sync_copy(hist_hbm, histall)

  carry = jnp.int32(0)
  for bb in range(NB // 16):
    tot = jnp.zeros((16,), jnp.int32)
    for w in range(NW):
      tot = tot + histall[pl.ds(_al(w * NB + bb * 16, 16), 16)]
    cntv[pl.ds(bb * 16, 16)] = tot
    a8 = (tot + 7) & ~7
    incl = plsc.cumsum(a8)
    basev[pl.ds(bb * 16, 16)] = carry + incl - a8
    carry = carry + jnp.sum(a8)

    def add_prev(w, s16):
      return s16 + histall[pl.ds(_al(w * NB + bb * 16, 16), 16)]

    mine = lax.fori_loop(0, wid, add_prev, basev[pl.ds(bb * 16, 16)])
    cur[pl.ds(bb * 16, 16)] = mine

  @pl.when(wid == 0)
  def _():
    pltpu.sync_copy(basev, bbase_hbm)
    pltpu.sync_copy(cntv, bcnt_hbm)

  pltpu.sync_copy(src_hbm.at[pl.ds(_al(wid * EW), WH)], srcw.at[pl.ds(0, WH)])
  pltpu.sync_copy(dst_hbm.at[pl.ds(_al(wid * EW), WH)], dstw.at[pl.ds(0, WH)])

  def win(w, carry2):
    par = lax.rem(w, 2)
    opp = 1 - par

    @pl.when(w + 1 < NWIN_H)
    def _():
      off = _al(wid * EW + (w + 1) * WH)
      pltpu.async_copy(src_hbm.at[pl.ds(off, WH)],
                       srcw.at[pl.ds(_al(opp * WH, 16), WH)], sem_in)
      pltpu.async_copy(dst_hbm.at[pl.ds(off, WH)],
                       dstw.at[pl.ds(_al(opp * WH, 16), WH)], sem_in)

    def vreg(j, cc):
      sv_ = srcw[pl.ds(_al(par * WH + j * 16, 16), 16)]
      dv = dstw[pl.ds(_al(par * WH + j * 16, 16), 16)]
      b = lax.div(dv, BW)
      dloc = dv - b * BW
      word = (sv_ << 10) | dloc
      sk, svl = plsc.sort_key_val(b, i16)
      tag16[...] = sk
      prev = plsc.load_gather(tag16, [jnp.maximum(i16 - 1, 0)])
      nxt = plsc.load_gather(tag16, [jnp.minimum(i16 + 1, 15)])
      first = (i16 == 0) | (sk != prev)
      last = (i16 == 15) | (sk != nxt)
      occ_s = i16 - plsc.cummax(jnp.where(first, i16, 0))
      base_s = plsc.load_gather(cur, [sk])
      pos_s = base_s + occ_s
      plsc.store_scatter(cur, [sk], pos_s + 1, mask=last)
      plsc.store_scatter(pos16, [svl], pos_s)
      wobuf[pl.ds(_al(par * WH + j * 16, 16), 16)] = word
      pobuf[par, pl.ds(_al(j * 16, 16), 16)] = pos16[...]
      return cc

    lax.fori_loop(0, WH // 16, vreg, 0)

    @pl.when(w > 0)
    def _():
      pltpu.make_async_copy(src_hbm.at[pl.ds(0, WH)],
                            wobuf.at[pl.ds(_al(opp * WH, 16), WH)], sem).wait()
    pltpu.async_copy(wobuf.at[pl.ds(_al(par * WH, 16), WH)],
                     words_hbm.at[pobuf.at[par]], sem)

    @pl.when(w + 1 < NWIN_H)
    def _():
      pltpu.make_async_copy(src_hbm.at[pl.ds(0, WH)],
                            srcw.at[pl.ds(_al(opp * WH, 16), WH)],
                            sem_in).wait()
      pltpu.make_async_copy(src_hbm.at[pl.ds(0, WH)],
                            dstw.at[pl.ds(_al(opp * WH, 16), WH)],
                            sem_in).wait()
    return carry2

  lax.fori_loop(0, NWIN_H, win, 0)
  pltpu.make_async_copy(src_hbm.at[pl.ds(0, WH)],
                        wobuf.at[pl.ds(((NWIN_H - 1) % 2) * WH, WH)],
                        sem).wait()


def _make_sc_edge(F, FP, with_cnt):
  BWF = BW * F
  n_out = 5 if with_cnt else 4
  outs = [jax.ShapeDtypeStruct((NP * F,), jnp.float32) for _ in range(4)]
  if with_cnt:
    outs.append(jax.ShapeDtypeStruct((NP,), jnp.float32))
  scratch = [
      pltpu.VMEM((2 * WE,), jnp.int32),
      pltpu.VMEM((2 * WE,), jnp.int32),
      pltpu.VMEM((2 * WE, FP), jnp.float32),
      pltpu.VMEM((BWF,), jnp.float32),
      pltpu.VMEM((BWF,), jnp.float32),
      pltpu.VMEM((BWF,), jnp.float32),
      pltpu.VMEM((BWF,), jnp.float32),
      pltpu.VMEM((BW,), jnp.float32),
      pltpu.VMEM((NB,), jnp.int32),
      pltpu.VMEM((NB,), jnp.int32),
      pltpu.VMEM((16,), jnp.int32),
      pltpu.VMEM((16,), jnp.int32),
      pltpu.SemaphoreType.DMA,
      pltpu.SemaphoreType.DMA,
  ]

  @functools.partial(
      pl.kernel, mesh=_mesh(), compiler_params=_SC_PARAMS,
      out_type=outs, scratch_types=scratch)
  def edge(words_hbm, bbase_hbm, bcnt_hbm, btab_hbm, zero_hbm, pinf_hbm,
           ninf_hbm, *rest):
    out_refs = rest[:n_out]
    (wbuf, idxb, brow, sacc, s2acc, mnacc, mxacc, cntacc, mbase, mcnt,
     tag16, occ16, sem_w, sem_g) = rest[n_out:]
    S_hbm, S2_hbm, Mn_hbm, Mx_hbm = out_refs[:4]
    cnt_hbm = out_refs[4] if with_cnt else None

    wid = _wid()
    i16 = _i16()
    onesf = jnp.ones((16,), jnp.float32)
    pltpu.sync_copy(bbase_hbm, mbase)
    pltpu.sync_copy(bcnt_hbm, mcnt)

    def scalar_at(ref, idx):
      chunk = ref[pl.ds(_al(lax.div(idx, 16) * 16, 16), 16)]
      return jnp.sum(jnp.where(i16 == lax.rem(idx, 16), chunk, 0))

    def build_idx(par):
      def bi(j, cc):
        v = wbuf[pl.ds(_al(par * WE + j * 16, 16), 16)]
        idxb[pl.ds(_al(par * WE + j * 16, 16), 16)] = jnp.clip(v >> 10, 0, N - 1)
        return cc

      lax.fori_loop(0, WE // 16, bi, 0)

    def issue_gather(par):
      for k in range(GC):
        pltpu.async_copy(btab_hbm.at[idxb.at[pl.ds(_al(par * WE + k * 128, 16), 128)]],
                         brow.at[pl.ds((par * GC + k) * 128, 128)], sem_g)

    def drain_gather(par):
      for k in range(GC):
        pltpu.make_async_copy(
            btab_hbm.at[pl.ds(0, 128)],
            brow.at[pl.ds((par * GC + k) * 128, 128)], sem_g).wait()

    def bin_body(p, carry0):
      b = wid * PASSES + p
      base = _al(scalar_at(mbase, b))
      cnt = scalar_at(mcnt, b)
      nwin = lax.div(cnt + (WE - 1), WE)
      pltpu.sync_copy(zero_hbm.at[pl.ds(0, BWF)], sacc)
      pltpu.sync_copy(zero_hbm.at[pl.ds(BWF, BWF)], s2acc)
      pltpu.sync_copy(pinf_hbm, mnacc)
      pltpu.sync_copy(ninf_hbm, mxacc)
      if with_cnt:
        pltpu.sync_copy(zero_hbm.at[pl.ds(0, BW)], cntacc)

      @pl.when(nwin > 0)
      def _():
        pltpu.sync_copy(words_hbm.at[pl.ds(base, WE)], wbuf.at[pl.ds(0, WE)])
        build_idx(0)
        issue_gather(0)

      @pl.when(nwin > 1)
      def _():
        pltpu.async_copy(words_hbm.at[pl.ds(_al(base + WE), WE)],
                         wbuf.at[pl.ds(WE, WE)], sem_w)

      def win_body(w, carry):
        par = lax.rem(w, 2)
        opp = 1 - par

        @pl.when(w + 1 < nwin)
        def _():
          pltpu.make_async_copy(words_hbm.at[pl.ds(0, WE)],
                                wbuf.at[pl.ds(_al(opp * WE, 16), WE)],
                                sem_w).wait()
          build_idx(opp)
          issue_gather(opp)

        drain_gather(par)

        def vreg(j, cc2):
          gpos = w * WE + j * 16 + i16
          valid = gpos < cnt
          v = wbuf[pl.ds(_al(par * WE + j * 16, 16), 16)]
          dloc = jnp.minimum(v & 1023, BW - 1)
          rowv = par * WE + j * 16 + i16
          dbase = dloc * F
          if with_cnt:
            plsc.addupdate_scatter(cntacc, [dloc], onesf, mask=valid)
          sk, sv2, om = plsc.sort_key_val(dloc, i16, mask=valid)
          tag16[...] = sk
          prev = plsc.load_gather(tag16, [jnp.maximum(i16 - 1, 0)])
          occ_s = i16 - plsc.cummax(
              jnp.where((i16 == 0) | (sk != prev), i16, 0))
          occ16[...] = jnp.zeros((16,), jnp.int32)
          plsc.store_scatter(occ16, [sv2], occ_s, mask=om)
          occ = occ16[...]
          ncls = jnp.max(jnp.where(om, occ_s, 0))
          mk0 = valid & (occ == 0)
          for f in range(F):
            fv = jnp.full((16,), f, jnp.int32)
            val = plsc.load_gather(brow, [rowv, fv], mask=valid)
            di = dbase + f
            plsc.addupdate_scatter(sacc, [di], val, mask=valid)
            plsc.addupdate_scatter(s2acc, [di], val * val, mask=valid)
            old = plsc.load_gather(mnacc, [di], mask=mk0)
            plsc.store_scatter(mnacc, [di], jnp.minimum(old, val), mask=mk0)
            old2 = plsc.load_gather(mxacc, [di], mask=mk0)
            plsc.store_scatter(mxacc, [di], jnp.maximum(old2, val), mask=mk0)

          def cls_body(kcls, cc):
            mk = valid & (occ == kcls)
            for f in range(F):
              fv = jnp.full((16,), f, jnp.int32)
              val = plsc.load_gather(brow, [rowv, fv], mask=mk)
              di = dbase + f
              old = plsc.load_gather(mnacc, [di], mask=mk)
              plsc.store_scatter(mnacc, [di], jnp.minimum(old, val), mask=mk)
              old2 = plsc.load_gather(mxacc, [di], mask=mk)
              plsc.store_scatter(mxacc, [di], jnp.maximum(old2, val), mask=mk)
            return cc

          lax.fori_loop(1, ncls + 1, cls_body, 0)
          return cc2

        lax.fori_loop(0, WE // 16, vreg, 0)

        @pl.when(w + 2 < nwin)
        def _():
          pltpu.async_copy(words_hbm.at[pl.ds(_al(base + (w + 2) * WE), WE)],
                           wbuf.at[pl.ds(_al(par * WE, 16), WE)], sem_w)

        return carry

      lax.fori_loop(0, nwin, win_body, 0)
      pltpu.sync_copy(sacc, S_hbm.at[pl.ds(_al(b * BWF), BWF)])
      pltpu.sync_copy(s2acc, S2_hbm.at[pl.ds(_al(b * BWF), BWF)])
      pltpu.sync_copy(mnacc, Mn_hbm.at[pl.ds(_al(b * BWF), BWF)])
      pltpu.sync_copy(mxacc, Mx_hbm.at[pl.ds(_al(b * BWF), BWF)])
      if with_cnt:
        pltpu.sync_copy(cntacc, cnt_hbm.at[pl.ds(_al(b * BW), BW)])
      return carry0

    lax.fori_loop(0, PASSES, bin_body, 0)

  return edge


_sc_edge1 = _make_sc_edge(25, 32, True)
_sc_edge2 = _make_sc_edge(16, 16, False)


_BLK0 = 2000
_BLK = 2048


def _full(spec_shape):
  return pl.BlockSpec(spec_shape, lambda i: (0, 0))


def _tc_pre1_body(x_ref, wsT_ref, b_ref, out_ref):
  bt = jnp.dot(x_ref[...], wsT_ref[...],
               preferred_element_type=jnp.float32) + b_ref[...]
  out_ref[...] = jnp.concatenate(
      [bt, jnp.zeros((bt.shape[0], 7), jnp.float32)], axis=1)


def _tc_pre1(x, Ws1T, bpre1):
  return pl.pallas_call(
      _tc_pre1_body,
      grid=(N // _BLK0,),
      in_specs=[pl.BlockSpec((_BLK0, 25), lambda i: (i, 0)),
                _full((25, 25)), _full((1, 25))],
      out_specs=pl.BlockSpec((_BLK0, 32), lambda i: (i, 0)),
      out_shape=jax.ShapeDtypeStruct((N, 32), jnp.float32),
  )(x, Ws1T, bpre1)


def _pna_combine(xb, A, S, S2, Mn, Mx, c, WxT, W1T, W2T, W3T, bpost):
  cg = c > 0.0
  cm = jnp.maximum(c, 1.0)
  sm = S / cm
  mean = jnp.where(cg, A + sm, 0.0)
  mn = jnp.where(cg, A + Mn, 0.0)
  mx = jnp.where(cg, A + Mx, 0.0)
  std = jnp.sqrt(jax.nn.relu(S2 / cm - sm * sm) + 1e-5)
  agg = jnp.concatenate([mean, mn, mx, std], axis=1)
  logd = jnp.log(cm + 1.0)
  amp = logd * (1.0 / AVG_DEG_LOG)
  att = AVG_DEG_LOG / logd
  out = (jnp.dot(xb, WxT, preferred_element_type=jnp.float32)
         + jnp.dot(agg, W1T, preferred_element_type=jnp.float32)
         + jnp.dot(agg * amp, W2T, preferred_element_type=jnp.float32)
         + jnp.dot(agg * att, W3T, preferred_element_type=jnp.float32)
         + bpost)
  return out


def _tc_post1_body(x_ref, S_ref, S2_ref, Mn_ref, Mx_ref, c_ref,
                   Wd1T_ref, Wx1T_ref, W11T_ref, W21T_ref, W31T_ref,
                   bpost1_ref, Wlin1T_ref, blin1_ref, Ws2T_ref, bpre2_ref,
                   h1_ref, b2tab_ref):
  xb = x_ref[...]
  A = jnp.dot(xb, Wd1T_ref[...], preferred_element_type=jnp.float32)
  out = _pna_combine(xb, A, S_ref[...], S2_ref[...], Mn_ref[...], Mx_ref[...],
                     c_ref[...], Wx1T_ref[...], W11T_ref[...], W21T_ref[...],
                     W31T_ref[...], bpost1_ref[...])
  h = jax.nn.relu(jnp.dot(out, Wlin1T_ref[...],
                          preferred_element_type=jnp.float32) + blin1_ref[...])
  h1_ref[...] = h
  b2tab_ref[...] = jnp.dot(h, Ws2T_ref[...],
                           preferred_element_type=jnp.float32) + bpre2_ref[...]


def _tc_post1(x_pad, S, S2, Mn, Mx, cnt, Wd1T, Wx1T, W11T, W21T, W31T,
              bpost1, Wlin1T, blin1, Ws2T, bpre2):
  nb = pl.BlockSpec((_BLK, 25), lambda i: (i, 0))
  return pl.pallas_call(
      _tc_post1_body,
      grid=(NP // _BLK,),
      in_specs=[nb, nb, nb, nb, nb,
                pl.BlockSpec((_BLK, 1), lambda i: (i, 0)),
                _full((25, 25)), _full((25, 16)), _full((100, 16)),
                _full((100, 16)), _full((100, 16)), _full((1, 16)),
                _full((16, 16)), _full((1, 16)), _full((16, 16)),
                _full((1, 16))],
      out_specs=[pl.BlockSpec((_BLK, 16), lambda i: (i, 0)),
                 pl.BlockSpec((_BLK, 16), lambda i: (i, 0))],
      out_shape=[jax.ShapeDtypeStruct((NP, 16), jnp.float32),
                 jax.ShapeDtypeStruct((NP, 16), jnp.float32)],
  )(x_pad, S, S2, Mn, Mx, cnt, Wd1T, Wx1T, W11T, W21T, W31T, bpost1,
    Wlin1T, blin1, Ws2T, bpre2)


def _tc_post2_body(h1_ref, S_ref, S2_ref, Mn_ref, Mx_ref, c_ref, batch_ref,
                   Wd2T_ref, Wx2T_ref, W12T_ref, W22T_ref, W32T_ref,
                   bpost2_ref, Wlin2T_ref, blin2_ref, WfcT_ref, bfc_ref,
                   out_ref, acc_ref):
  i = pl.program_id(0)
  hb = h1_ref[...]
  A = jnp.dot(hb, Wd2T_ref[...], preferred_element_type=jnp.float32)
  out = _pna_combine(hb, A, S_ref[...], S2_ref[...], Mn_ref[...], Mx_ref[...],
                     c_ref[...], Wx2T_ref[...], W12T_ref[...], W22T_ref[...],
                     W32T_ref[...], bpost2_ref[...])
  h2 = jax.nn.relu(jnp.dot(out, Wlin2T_ref[...],
                           preferred_element_type=jnp.float32) + blin2_ref[...])
  rows = i * _BLK + lax.broadcasted_iota(jnp.int32, (_BLK, 1), 0)
  rmask = rows < N
  h2 = jnp.where(rmask, h2, 0.0)
  onehot = ((batch_ref[...] == lax.broadcasted_iota(jnp.int32, (_BLK, G), 1))
            & rmask).astype(jnp.float32)
  contrib = lax.dot_general(onehot, h2, (((0,), (0,)), ((), ())),
                            preferred_element_type=jnp.float32)

  @pl.when(i == 0)
  def _():
    acc_ref[...] = jnp.zeros_like(acc_ref)

  acc_ref[...] += contrib

  @pl.when(i == NP // _BLK - 1)
  def _():
    logits = jnp.dot(acc_ref[...], WfcT_ref[...],
                     preferred_element_type=jnp.float32) + bfc_ref[...]
    m = jnp.max(logits, axis=1, keepdims=True)
    ex = jnp.exp(logits - m)
    out_ref[...] = (logits - m) - jnp.log(jnp.sum(ex, axis=1, keepdims=True))


def _tc_post2(h1, S, S2, Mn, Mx, cnt, batch_pad, Wd2T, Wx2T, W12T, W22T, W32T,
              bpost2, Wlin2T, blin2, WfcT, bfc):
  nb = pl.BlockSpec((_BLK, 16), lambda i: (i, 0))
  cb = pl.BlockSpec((_BLK, 1), lambda i: (i, 0))
  return pl.pallas_call(
      _tc_post2_body,
      grid=(NP // _BLK,),
      in_specs=[nb, nb, nb, nb, nb, cb, cb,
                _full((16, 16)), _full((16, 8)), _full((64, 8)),
                _full((64, 8)), _full((64, 8)), _full((1, 8)),
                _full((8, 8)), _full((1, 8)), _full((8, 2)), _full((1, 2))],
      out_specs=pl.BlockSpec((G, 2), lambda i: (0, 0)),
      out_shape=jax.ShapeDtypeStruct((G, 2), jnp.float32),
      scratch_shapes=[pltpu.VMEM((G, 8), jnp.float32)],
  )(h1, S, S2, Mn, Mx, cnt, batch_pad, Wd2T, Wx2T, W12T, W22T, W32T, bpost2,
    Wlin2T, blin2, WfcT, bfc)


def kernel(x, edge_index, batch,
           Wpre1, bpre1, Wpost1, bpost1, Wlin1, blin1,
           Wpre2, bpre2, Wpost2, bpost2, Wlin2, blin2,
           Wfc, bfc):
  src = edge_index[0]
  dst = edge_index[1]

  hist = _sc_hist(dst)
  bwords, bbase, bcnt = _sc_permute(src, dst, hist)

  Wd1T = Wpre1[:, :25].T
  Ws1T = Wpre1[:, 25:].T
  Wx1T = Wpost1[:, :25].T
  W11T = Wpost1[:, 25:125].T
  W21T = Wpost1[:, 125:225].T
  W31T = Wpost1[:, 225:325].T
  Wd2T = Wpre2[:, :16].T
  Ws2T = Wpre2[:, 16:].T
  Wx2T = Wpost2[:, :16].T
  W12T = Wpost2[:, 16:80].T
  W22T = Wpost2[:, 80:144].T
  W32T = Wpost2[:, 144:208].T

  BWF1 = BW * 25
  zero_t = jnp.zeros((2 * BWF1,), jnp.float32)
  pinf_t = jnp.full((BWF1,), jnp.inf, jnp.float32)
  ninf_t = jnp.full((BWF1,), -jnp.inf, jnp.float32)
  zero_t2 = jnp.zeros((2 * BW * 16,), jnp.float32)
  pinf_t2 = jnp.full((BW * 16,), jnp.inf, jnp.float32)
  ninf_t2 = jnp.full((BW * 16,), -jnp.inf, jnp.float32)

  b1tab = _tc_pre1(x, Ws1T, bpre1.reshape(1, 25))
  S1, S21, Mn1, Mx1, cntf = _sc_edge1(bwords, bbase, bcnt, b1tab,
                                      zero_t, pinf_t, ninf_t)
  x_pad = jnp.pad(x, ((0, NP - N), (0, 0)))
  cnt2d = cntf.reshape(NP, 1)
  h1, b2tab = _tc_post1(
      x_pad, S1.reshape(NP, 25), S21.reshape(NP, 25), Mn1.reshape(NP, 25),
      Mx1.reshape(NP, 25), cnt2d, Wd1T, Wx1T, W11T, W21T, W31T,
      bpost1.reshape(1, 16), Wlin1.T, blin1.reshape(1, 16), Ws2T,
      bpre2.reshape(1, 16))

  S2_, S22, Mn2, Mx2 = _sc_edge2(bwords, bbase, bcnt, b2tab,
                                 zero_t2, pinf_t2, ninf_t2)
  batch_pad = jnp.pad(batch, (0, NP - N)).reshape(NP, 1)
  out = _tc_post2(
      h1, S2_.reshape(NP, 16), S22.reshape(NP, 16), Mn2.reshape(NP, 16),
      Mx2.reshape(NP, 16), cnt2d, batch_pad, Wd2T, Wx2T, W12T, W22T, W32T,
      bpost2.reshape(1, 8), Wlin2.T, blin2.reshape(1, 8), Wfc.T,
      bfc.reshape(1, 2))
  return out


# speedup vs baseline: 6.4741x; 1.0148x over previous
import functools
import numpy as np
import jax
import jax.numpy as jnp
from jax import lax
from jax.experimental import pallas as pl
from jax.experimental.pallas import tpu as pltpu
from jax.experimental.pallas import tpu_sc as plsc

N = 100000
E = 1600000
G = 128
AVG_DEG_LOG = float(np.log(17.0))

NC = 2
NS = 16
NW = NC * NS
BW = 784
NB = 128
NP = BW * NB
PASSES = NB // NW

EW = E // NW
WH = 2000
NWIN_H = EW // WH

EP = 1602048
TRASH = EP - 64

WE = 512
GC = WE // 128

_SC_PARAMS = pltpu.CompilerParams(
    needs_layout_passes=False, use_tc_tiling_on_sc=False)


def _mesh():
  return plsc.VectorSubcoreMesh(core_axis_name="c", subcore_axis_name="s")


def _wid():
  return lax.axis_index("s") * NC + lax.axis_index("c")


def _i16():
  return lax.iota(jnp.int32, 16)


def _al(v, m=8):
  return pl.multiple_of(v, m)


@functools.partial(
    pl.kernel, mesh=_mesh(), compiler_params=_SC_PARAMS,
    out_type=jax.ShapeDtypeStruct((NW * NB,), jnp.int32),
    scratch_types=[pltpu.VMEM((2 * WH,), jnp.int32),
                   pltpu.VMEM((NB,), jnp.int32),
                   pltpu.SemaphoreType.DMA],
)
def _sc_hist(dst_hbm, hist_hbm, dstw, histv, sem):
  wid = _wid()
  for j in range(NB // 16):
    histv[pl.ds(j * 16, 16)] = jnp.zeros((16,), jnp.int32)
  ones = jnp.ones((16,), jnp.int32)
  pltpu.sync_copy(dst_hbm.at[pl.ds(_al(wid * EW), WH)], dstw.at[pl.ds(0, WH)])

  def win(w, carry):
    par = lax.rem(w, 2)
    opp = 1 - par

    @pl.when(w + 1 < NWIN_H)
    def _():
      pltpu.async_copy(dst_hbm.at[pl.ds(_al(wid * EW + (w + 1) * WH), WH)],
                       dstw.at[pl.ds(_al(opp * WH, 16), WH)], sem)

    def vreg(j, cc):
      v = dstw[pl.ds(_al(par * WH + j * 16, 16), 16)]
      b = lax.div(v, BW)
      plsc.addupdate_scatter(histv, [b], ones)
      return cc

    lax.fori_loop(0, WH // 16, vreg, 0)

    @pl.when(w + 1 < NWIN_H)
    def _():
      pltpu.make_async_copy(dst_hbm.at[pl.ds(0, WH)],
                            dstw.at[pl.ds(_al(opp * WH, 16), WH)], sem).wait()
    return carry

  lax.fori_loop(0, NWIN_H, win, 0)
  pltpu.sync_copy(histv, hist_hbm.at[pl.ds(_al(wid * NB), NB)])


@functools.partial(
    pl.kernel, mesh=_mesh(), compiler_params=_SC_PARAMS,
    out_type=[jax.ShapeDtypeStruct((EP,), jnp.int32),
              jax.ShapeDtypeStruct((NB,), jnp.int32),
              jax.ShapeDtypeStruct((NB,), jnp.int32)],
    scratch_types=[pltpu.VMEM((2 * WH,), jnp.int32),
                   pltpu.VMEM((2 * WH,), jnp.int32),
                   pltpu.VMEM((NW * NB,), jnp.int32),
                   pltpu.VMEM((NB,), jnp.int32),
                   pltpu.VMEM((NB,), jnp.int32),
                   pltpu.VMEM((NB,), jnp.int32),
                   pltpu.VMEM((2 * WH,), jnp.int32),
                   pltpu.VMEM((2, WH), jnp.int32),
                   pltpu.VMEM((16,), jnp.int32),
                   pltpu.VMEM((16,), jnp.int32),
                   pltpu.SemaphoreType.DMA,
                   pltpu.SemaphoreType.DMA],
)
def _sc_permute(src_hbm, dst_hbm, hist_hbm, words_hbm, bbase_hbm, bcnt_hbm,
                srcw, dstw, histall, cntv, basev, cur, wobuf, pobuf,
                tag16, pos16, sem, sem_in):
  wid = _wid()
  i16 = _i16()
  pltpu.sync_copy(hist_hbm, histall)

  carry = jnp.int32(0)
  for bb in range(NB // 16):
    tot = jnp.zeros((16,), jnp.int32)
    for w in range(NW):
      tot = tot + histall[pl.ds(_al(w * NB + bb * 16, 16), 16)]
    cntv[pl.ds(bb * 16, 16)] = tot
    a8 = (tot + 7) & ~7
    incl = plsc.cumsum(a8)
    basev[pl.ds(bb * 16, 16)] = carry + incl - a8
    carry = carry + jnp.sum(a8)

    def add_prev(w, s16):
      return s16 + histall[pl.ds(_al(w * NB + bb * 16, 16), 16)]

    mine = lax.fori_loop(0, wid, add_prev, basev[pl.ds(bb * 16, 16)])
    cur[pl.ds(bb * 16, 16)] = mine

  @pl.when(wid == 0)
  def _():
    pltpu.sync_copy(basev, bbase_hbm)
    pltpu.sync_copy(cntv, bcnt_hbm)

  pltpu.sync_copy(src_hbm.at[pl.ds(_al(wid * EW), WH)], srcw.at[pl.ds(0, WH)])
  pltpu.sync_copy(dst_hbm.at[pl.ds(_al(wid * EW), WH)], dstw.at[pl.ds(0, WH)])

  def win(w, carry2):
    par = lax.rem(w, 2)
    opp = 1 - par

    @pl.when(w + 1 < NWIN_H)
    def _():
      off = _al(wid * EW + (w + 1) * WH)
      pltpu.async_copy(src_hbm.at[pl.ds(off, WH)],
                       srcw.at[pl.ds(_al(opp * WH, 16), WH)], sem_in)
      pltpu.async_copy(dst_hbm.at[pl.ds(off, WH)],
                       dstw.at[pl.ds(_al(opp * WH, 16), WH)], sem_in)

    def vreg(j, cc):
      sv_ = srcw[pl.ds(_al(par * WH + j * 16, 16), 16)]
      dv = dstw[pl.ds(_al(par * WH + j * 16, 16), 16)]
      b = lax.div(dv, BW)
      dloc = dv - b * BW
      word = (sv_ << 10) | dloc
      sk, svl = plsc.sort_key_val(b, i16)
      tag16[...] = sk
      prev = plsc.load_gather(tag16, [jnp.maximum(i16 - 1, 0)])
      nxt = plsc.load_gather(tag16, [jnp.minimum(i16 + 1, 15)])
      first = (i16 == 0) | (sk != prev)
      last = (i16 == 15) | (sk != nxt)
      occ_s = i16 - plsc.cummax(jnp.where(first, i16, 0))
      base_s = plsc.load_gather(cur, [sk])
      pos_s = base_s + occ_s
      plsc.store_scatter(cur, [sk], pos_s + 1, mask=last)
      plsc.store_scatter(pos16, [svl], pos_s)
      wobuf[pl.ds(_al(par * WH + j * 16, 16), 16)] = word
      pobuf[par, pl.ds(_al(j * 16, 16), 16)] = pos16[...]
      return cc

    lax.fori_loop(0, WH // 16, vreg, 0)

    @pl.when(w > 0)
    def _():
      pltpu.make_async_copy(src_hbm.at[pl.ds(0, WH)],
                            wobuf.at[pl.ds(_al(opp * WH, 16), WH)], sem).wait()
    pltpu.async_copy(wobuf.at[pl.ds(_al(par * WH, 16), WH)],
                     words_hbm.at[pobuf.at[par]], sem)

    @pl.when(w + 1 < NWIN_H)
    def _():
      pltpu.make_async_copy(src_hbm.at[pl.ds(0, WH)],
                            srcw.at[pl.ds(_al(opp * WH, 16), WH)],
                            sem_in).wait()
      pltpu.make_async_copy(src_hbm.at[pl.ds(0, WH)],
                            dstw.at[pl.ds(_al(opp * WH, 16), WH)],
                            sem_in).wait()
    return carry2

  lax.fori_loop(0, NWIN_H, win, 0)
  pltpu.make_async_copy(src_hbm.at[pl.ds(0, WH)],
                        wobuf.at[pl.ds(((NWIN_H - 1) % 2) * WH, WH)],
                        sem).wait()


def _make_sc_edge(F, FP, with_cnt):
  BWF = BW * F
  n_out = 5 if with_cnt else 4
  outs = [jax.ShapeDtypeStruct((NP * F,), jnp.float32) for _ in range(4)]
  if with_cnt:
    outs.append(jax.ShapeDtypeStruct((NP,), jnp.float32))
  scratch = [
      pltpu.VMEM((2 * WE,), jnp.int32),
      pltpu.VMEM((2 * WE,), jnp.int32),
      pltpu.VMEM((2 * WE, FP), jnp.float32),
      pltpu.VMEM((BWF,), jnp.float32),
      pltpu.VMEM((BWF,), jnp.float32),
      pltpu.VMEM((BWF,), jnp.float32),
      pltpu.VMEM((BWF,), jnp.float32),
      pltpu.VMEM((BW,), jnp.float32),
      pltpu.VMEM((NB,), jnp.int32),
      pltpu.VMEM((NB,), jnp.int32),
      pltpu.VMEM((16,), jnp.int32),
      pltpu.VMEM((16,), jnp.int32),
      pltpu.SemaphoreType.DMA,
      pltpu.SemaphoreType.DMA,
  ]

  @functools.partial(
      pl.kernel, mesh=_mesh(), compiler_params=_SC_PARAMS,
      out_type=outs, scratch_types=scratch)
  def edge(words_hbm, bbase_hbm, bcnt_hbm, btab_hbm, zero_hbm, pinf_hbm,
           ninf_hbm, *rest):
    out_refs = rest[:n_out]
    (wbuf, idxb, brow, sacc, s2acc, mnacc, mxacc, cntacc, mbase, mcnt,
     tag16, occ16, sem_w, sem_g) = rest[n_out:]
    S_hbm, S2_hbm, Mn_hbm, Mx_hbm = out_refs[:4]
    cnt_hbm = out_refs[4] if with_cnt else None

    wid = _wid()
    i16 = _i16()
    onesf = jnp.ones((16,), jnp.float32)
    pltpu.sync_copy(bbase_hbm, mbase)
    pltpu.sync_copy(bcnt_hbm, mcnt)

    def scalar_at(ref, idx):
      chunk = ref[pl.ds(_al(lax.div(idx, 16) * 16, 16), 16)]
      return jnp.sum(jnp.where(i16 == lax.rem(idx, 16), chunk, 0))

    def build_idx(par):
      def bi(j, cc):
        v = wbuf[pl.ds(_al(par * WE + j * 16, 16), 16)]
        idxb[pl.ds(_al(par * WE + j * 16, 16), 16)] = jnp.clip(v >> 10, 0, N - 1)
        return cc

      lax.fori_loop(0, WE // 16, bi, 0)

    def issue_gather(par):
      for k in range(GC):
        pltpu.async_copy(btab_hbm.at[idxb.at[pl.ds(_al(par * WE + k * 128, 16), 128)]],
                         brow.at[pl.ds((par * GC + k) * 128, 128)], sem_g)

    def drain_gather(par):
      for k in range(GC):
        pltpu.make_async_copy(
            btab_hbm.at[pl.ds(0, 128)],
            brow.at[pl.ds((par * GC + k) * 128, 128)], sem_g).wait()

    def bin_body(p, carry0):
      b = wid * PASSES + p
      base = _al(scalar_at(mbase, b))
      cnt = scalar_at(mcnt, b)
      nwin = lax.div(cnt + (WE - 1), WE)
      pltpu.sync_copy(zero_hbm.at[pl.ds(0, BWF)], sacc)
      pltpu.sync_copy(zero_hbm.at[pl.ds(BWF, BWF)], s2acc)
      pltpu.sync_copy(pinf_hbm, mnacc)
      pltpu.sync_copy(ninf_hbm, mxacc)
      if with_cnt:
        pltpu.sync_copy(zero_hbm.at[pl.ds(0, BW)], cntacc)

      @pl.when(nwin > 0)
      def _():
        pltpu.sync_copy(words_hbm.at[pl.ds(base, WE)], wbuf.at[pl.ds(0, WE)])
        build_idx(0)
        issue_gather(0)

      @pl.when(nwin > 1)
      def _():
        pltpu.async_copy(words_hbm.at[pl.ds(_al(base + WE), WE)],
                         wbuf.at[pl.ds(WE, WE)], sem_w)

      def win_body(w, carry):
        par = lax.rem(w, 2)
        opp = 1 - par

        @pl.when(w + 1 < nwin)
        def _():
          pltpu.make_async_copy(words_hbm.at[pl.ds(0, WE)],
                                wbuf.at[pl.ds(_al(opp * WE, 16), WE)],
                                sem_w).wait()
          build_idx(opp)
          issue_gather(opp)

        drain_gather(par)

        def vreg(j, cc2):
          gpos = w * WE + j * 16 + i16
          valid = gpos < cnt
          v = wbuf[pl.ds(_al(par * WE + j * 16, 16), 16)]
          dloc = jnp.minimum(v & 1023, BW - 1)
          rowv = par * WE + j * 16 + i16
          dbase = dloc * F
          if with_cnt:
            plsc.addupdate_scatter(cntacc, [dloc], onesf, mask=valid)
          plsc.store_scatter(tag16, [dloc], i16, mask=valid)
          rd = plsc.load_gather(tag16, [dloc], mask=valid)
          win = valid & (rd == i16)
          for f in range(F):
            fv = jnp.full((16,), f, jnp.int32)
            val = plsc.load_gather(brow, [rowv, fv], mask=valid)
            di = dbase + f
            plsc.addupdate_scatter(sacc, [di], val, mask=valid)
            plsc.addupdate_scatter(s2acc, [di], val * val, mask=valid)
            old = plsc.load_gather(mnacc, [di], mask=win)
            plsc.store_scatter(mnacc, [di], jnp.minimum(old, val), mask=win)
            old2 = plsc.load_gather(mxacc, [di], mask=win)
            plsc.store_scatter(mxacc, [di], jnp.maximum(old2, val), mask=win)

          def peel_cond(remi):
            return jnp.max(remi) > 0

          def peel_body(remi):
            rem = remi > 0
            plsc.store_scatter(tag16, [dloc], i16, mask=rem)
            rd2 = plsc.load_gather(tag16, [dloc], mask=rem)
            w = rem & (rd2 == i16)
            for f in range(F):
              fv = jnp.full((16,), f, jnp.int32)
              val = plsc.load_gather(brow, [rowv, fv], mask=w)
              di = dbase + f
              old = plsc.load_gather(mnacc, [di], mask=w)
              plsc.store_scatter(mnacc, [di], jnp.minimum(old, val), mask=w)
              old2 = plsc.load_gather(mxacc, [di], mask=w)
              plsc.store_scatter(mxacc, [di], jnp.maximum(old2, val), mask=w)
            return jnp.where(w, 0, remi)

          lax.while_loop(peel_cond, peel_body,
                         jnp.where(valid & ~win, 1, 0))
          return cc2

        lax.fori_loop(0, WE // 16, vreg, 0)

        @pl.when(w + 2 < nwin)
        def _():
          pltpu.async_copy(words_hbm.at[pl.ds(_al(base + (w + 2) * WE), WE)],
                           wbuf.at[pl.ds(_al(par * WE, 16), WE)], sem_w)

        return carry

      lax.fori_loop(0, nwin, win_body, 0)
      pltpu.sync_copy(sacc, S_hbm.at[pl.ds(_al(b * BWF), BWF)])
      pltpu.sync_copy(s2acc, S2_hbm.at[pl.ds(_al(b * BWF), BWF)])
      pltpu.sync_copy(mnacc, Mn_hbm.at[pl.ds(_al(b * BWF), BWF)])
      pltpu.sync_copy(mxacc, Mx_hbm.at[pl.ds(_al(b * BWF), BWF)])
      if with_cnt:
        pltpu.sync_copy(cntacc, cnt_hbm.at[pl.ds(_al(b * BW), BW)])
      return carry0

    lax.fori_loop(0, PASSES, bin_body, 0)

  return edge


_sc_edge1 = _make_sc_edge(25, 32, True)
_sc_edge2 = _make_sc_edge(16, 16, False)


_BLK0 = 2000
_BLK = 2048


def _full(spec_shape):
  return pl.BlockSpec(spec_shape, lambda i: (0, 0))


def _tc_pre1_body(x_ref, wsT_ref, b_ref, out_ref):
  bt = jnp.dot(x_ref[...], wsT_ref[...],
               preferred_element_type=jnp.float32) + b_ref[...]
  out_ref[...] = jnp.concatenate(
      [bt, jnp.zeros((bt.shape[0], 7), jnp.float32)], axis=1)


def _tc_pre1(x, Ws1T, bpre1):
  return pl.pallas_call(
      _tc_pre1_body,
      grid=(N // _BLK0,),
      in_specs=[pl.BlockSpec((_BLK0, 25), lambda i: (i, 0)),
                _full((25, 25)), _full((1, 25))],
      out_specs=pl.BlockSpec((_BLK0, 32), lambda i: (i, 0)),
      out_shape=jax.ShapeDtypeStruct((N, 32), jnp.float32),
  )(x, Ws1T, bpre1)


def _pna_combine(xb, A, S, S2, Mn, Mx, c, WxT, W1T, W2T, W3T, bpost):
  cg = c > 0.0
  cm = jnp.maximum(c, 1.0)
  sm = S / cm
  mean = jnp.where(cg, A + sm, 0.0)
  mn = jnp.where(cg, A + Mn, 0.0)
  mx = jnp.where(cg, A + Mx, 0.0)
  std = jnp.sqrt(jax.nn.relu(S2 / cm - sm * sm) + 1e-5)
  agg = jnp.concatenate([mean, mn, mx, std], axis=1)
  logd = jnp.log(cm + 1.0)
  amp = logd * (1.0 / AVG_DEG_LOG)
  att = AVG_DEG_LOG / logd
  out = (jnp.dot(xb, WxT, preferred_element_type=jnp.float32)
         + jnp.dot(agg, W1T, preferred_element_type=jnp.float32)
         + jnp.dot(agg * amp, W2T, preferred_element_type=jnp.float32)
         + jnp.dot(agg * att, W3T, preferred_element_type=jnp.float32)
         + bpost)
  return out


def _tc_post1_body(x_ref, S_ref, S2_ref, Mn_ref, Mx_ref, c_ref,
                   Wd1T_ref, Wx1T_ref, W11T_ref, W21T_ref, W31T_ref,
                   bpost1_ref, Wlin1T_ref, blin1_ref, Ws2T_ref, bpre2_ref,
                   h1_ref, b2tab_ref):
  xb = x_ref[...]
  A = jnp.dot(xb, Wd1T_ref[...], preferred_element_type=jnp.float32)
  out = _pna_combine(xb, A, S_ref[...], S2_ref[...], Mn_ref[...], Mx_ref[...],
                     c_ref[...], Wx1T_ref[...], W11T_ref[...], W21T_ref[...],
                     W31T_ref[...], bpost1_ref[...])
  h = jax.nn.relu(jnp.dot(out, Wlin1T_ref[...],
                          preferred_element_type=jnp.float32) + blin1_ref[...])
  h1_ref[...] = h
  b2tab_ref[...] = jnp.dot(h, Ws2T_ref[...],
                           preferred_element_type=jnp.float32) + bpre2_ref[...]


def _tc_post1(x_pad, S, S2, Mn, Mx, cnt, Wd1T, Wx1T, W11T, W21T, W31T,
              bpost1, Wlin1T, blin1, Ws2T, bpre2):
  nb = pl.BlockSpec((_BLK, 25), lambda i: (i, 0))
  return pl.pallas_call(
      _tc_post1_body,
      grid=(NP // _BLK,),
      in_specs=[nb, nb, nb, nb, nb,
                pl.BlockSpec((_BLK, 1), lambda i: (i, 0)),
                _full((25, 25)), _full((25, 16)), _full((100, 16)),
                _full((100, 16)), _full((100, 16)), _full((1, 16)),
                _full((16, 16)), _full((1, 16)), _full((16, 16)),
                _full((1, 16))],
      out_specs=[pl.BlockSpec((_BLK, 16), lambda i: (i, 0)),
                 pl.BlockSpec((_BLK, 16), lambda i: (i, 0))],
      out_shape=[jax.ShapeDtypeStruct((NP, 16), jnp.float32),
                 jax.ShapeDtypeStruct((NP, 16), jnp.float32)],
  )(x_pad, S, S2, Mn, Mx, cnt, Wd1T, Wx1T, W11T, W21T, W31T, bpost1,
    Wlin1T, blin1, Ws2T, bpre2)


def _tc_post2_body(h1_ref, S_ref, S2_ref, Mn_ref, Mx_ref, c_ref, batch_ref,
                   Wd2T_ref, Wx2T_ref, W12T_ref, W22T_ref, W32T_ref,
                   bpost2_ref, Wlin2T_ref, blin2_ref, WfcT_ref, bfc_ref,
                   out_ref, acc_ref):
  i = pl.program_id(0)
  hb = h1_ref[...]
  A = jnp.dot(hb, Wd2T_ref[...], preferred_element_type=jnp.float32)
  out = _pna_combine(hb, A, S_ref[...], S2_ref[...], Mn_ref[...], Mx_ref[...],
                     c_ref[...], Wx2T_ref[...], W12T_ref[...], W22T_ref[...],
                     W32T_ref[...], bpost2_ref[...])
  h2 = jax.nn.relu(jnp.dot(out, Wlin2T_ref[...],
                           preferred_element_type=jnp.float32) + blin2_ref[...])
  rows = i * _BLK + lax.broadcasted_iota(jnp.int32, (_BLK, 1), 0)
  rmask = rows < N
  h2 = jnp.where(rmask, h2, 0.0)
  onehot = ((batch_ref[...] == lax.broadcasted_iota(jnp.int32, (_BLK, G), 1))
            & rmask).astype(jnp.float32)
  contrib = lax.dot_general(onehot, h2, (((0,), (0,)), ((), ())),
                            preferred_element_type=jnp.float32)

  @pl.when(i == 0)
  def _():
    acc_ref[...] = jnp.zeros_like(acc_ref)

  acc_ref[...] += contrib

  @pl.when(i == NP // _BLK - 1)
  def _():
    logits = jnp.dot(acc_ref[...], WfcT_ref[...],
                     preferred_element_type=jnp.float32) + bfc_ref[...]
    m = jnp.max(logits, axis=1, keepdims=True)
    ex = jnp.exp(logits - m)
    out_ref[...] = (logits - m) - jnp.log(jnp.sum(ex, axis=1, keepdims=True))


def _tc_post2(h1, S, S2, Mn, Mx, cnt, batch_pad, Wd2T, Wx2T, W12T, W22T, W32T,
              bpost2, Wlin2T, blin2, WfcT, bfc):
  nb = pl.BlockSpec((_BLK, 16), lambda i: (i, 0))
  cb = pl.BlockSpec((_BLK, 1), lambda i: (i, 0))
  return pl.pallas_call(
      _tc_post2_body,
      grid=(NP // _BLK,),
      in_specs=[nb, nb, nb, nb, nb, cb, cb,
                _full((16, 16)), _full((16, 8)), _full((64, 8)),
                _full((64, 8)), _full((64, 8)), _full((1, 8)),
                _full((8, 8)), _full((1, 8)), _full((8, 2)), _full((1, 2))],
      out_specs=pl.BlockSpec((G, 2), lambda i: (0, 0)),
      out_shape=jax.ShapeDtypeStruct((G, 2), jnp.float32),
      scratch_shapes=[pltpu.VMEM((G, 8), jnp.float32)],
  )(h1, S, S2, Mn, Mx, cnt, batch_pad, Wd2T, Wx2T, W12T, W22T, W32T, bpost2,
    Wlin2T, blin2, WfcT, bfc)


def kernel(x, edge_index, batch,
           Wpre1, bpre1, Wpost1, bpost1, Wlin1, blin1,
           Wpre2, bpre2, Wpost2, bpost2, Wlin2, blin2,
           Wfc, bfc):
  src = edge_index[0]
  dst = edge_index[1]

  hist = _sc_hist(dst)
  bwords, bbase, bcnt = _sc_permute(src, dst, hist)

  Wd1T = Wpre1[:, :25].T
  Ws1T = Wpre1[:, 25:].T
  Wx1T = Wpost1[:, :25].T
  W11T = Wpost1[:, 25:125].T
  W21T = Wpost1[:, 125:225].T
  W31T = Wpost1[:, 225:325].T
  Wd2T = Wpre2[:, :16].T
  Ws2T = Wpre2[:, 16:].T
  Wx2T = Wpost2[:, :16].T
  W12T = Wpost2[:, 16:80].T
  W22T = Wpost2[:, 80:144].T
  W32T = Wpost2[:, 144:208].T

  BWF1 = BW * 25
  zero_t = jnp.zeros((2 * BWF1,), jnp.float32)
  pinf_t = jnp.full((BWF1,), jnp.inf, jnp.float32)
  ninf_t = jnp.full((BWF1,), -jnp.inf, jnp.float32)
  zero_t2 = jnp.zeros((2 * BW * 16,), jnp.float32)
  pinf_t2 = jnp.full((BW * 16,), jnp.inf, jnp.float32)
  ninf_t2 = jnp.full((BW * 16,), -jnp.inf, jnp.float32)

  b1tab = _tc_pre1(x, Ws1T, bpre1.reshape(1, 25))
  S1, S21, Mn1, Mx1, cntf = _sc_edge1(bwords, bbase, bcnt, b1tab,
                                      zero_t, pinf_t, ninf_t)
  x_pad = jnp.pad(x, ((0, NP - N), (0, 0)))
  cnt2d = cntf.reshape(NP, 1)
  h1, b2tab = _tc_post1(
      x_pad, S1.reshape(NP, 25), S21.reshape(NP, 25), Mn1.reshape(NP, 25),
      Mx1.reshape(NP, 25), cnt2d, Wd1T, Wx1T, W11T, W21T, W31T,
      bpost1.reshape(1, 16), Wlin1.T, blin1.reshape(1, 16), Ws2T,
      bpre2.reshape(1, 16))

  S2_, S22, Mn2, Mx2 = _sc_edge2(bwords, bbase, bcnt, b2tab,
                                 zero_t2, pinf_t2, ninf_t2)
  batch_pad = jnp.pad(batch, (0, NP - N)).reshape(NP, 1)
  out = _tc_post2(
      h1, S2_.reshape(NP, 16), S22.reshape(NP, 16), Mn2.reshape(NP, 16),
      Mx2.reshape(NP, 16), cnt2d, batch_pad, Wd2T, Wx2T, W12T, W22T, W32T,
      bpost2.reshape(1, 8), Wlin2.T, blin2.reshape(1, 8), Wfc.T,
      bfc.reshape(1, 2))
  return out
